# baseline (speedup 1.0000x reference)
import functools

import jax
import jax.numpy as jnp
from jax import lax
from jax.experimental import pallas as pl
from jax.experimental.pallas import tpu as pltpu
from jax.experimental.pallas import tpu_sc as plsc

N = 10000
NP = 10240
E = 160000
EP = 163840
D = 256
H = 256
HH = H // 2
NC, NS = 2, 16
CW = 128
EPT = EP // NS
CH = EPT // CW
CHD = EPT // NC // CW
RPT = NP // NS
RB = 1000
NBLK = N // RB


def _sc_mesh():
    return plsc.VectorSubcoreMesh(core_axis_name="c", subcore_axis_name="s")


NBUF = 2
GRP = 8


@functools.partial(
    pl.kernel,
    out_type=jax.ShapeDtypeStruct((NC, NP, HH), jnp.float32),
    mesh=_sc_mesh(),
    scratch_types=(
        [pltpu.VMEM((GRP, CW), jnp.int32)] * 2
        + [pltpu.VMEM((CW,), jnp.int32)] * (2 * NBUF)
        + [pltpu.VMEM((CW, HH), jnp.float32)] * NBUF
        + [pltpu.VMEM_SHARED((NP, HH), jnp.float32)]
        + [pltpu.SemaphoreType.DMA] * (2 * NBUF)
    ),
)
def _conv_kernel(hs_hbm, src_hbm, dst_hbm, out_hbm, sring, dring, *rest):
    scur = rest[0:NBUF]
    dcur = rest[NBUF:2 * NBUF]
    gbuf = rest[2 * NBUF:3 * NBUF]
    acc = rest[3 * NBUF]
    gsem = rest[3 * NBUF + 1:3 * NBUF + 1 + NBUF]
    ssem = rest[3 * NBUF + 1 + NBUF:3 * NBUF + 1 + 2 * NBUF]
    c = lax.axis_index("c")
    s = lax.axis_index("s")
    pltpu.sync_copy(hs_hbm.at[c, pl.ds(s * RPT, RPT)], acc.at[pl.ds(s * RPT, RPT)])
    plsc.subcore_barrier()
    table = hs_hbm.at[c]

    def group(gg, carry):
        pltpu.sync_copy(src_hbm.at[s, pl.ds(gg * GRP, GRP)], sring)
        pltpu.sync_copy(dst_hbm.at[s, pl.ds(gg * GRP, GRP)], dring)
        gds = [None] * NBUF
        sds = [None] * NBUF
        for kk in range(GRP):
            b = kk % NBUF
            if sds[b] is not None:
                sds[b].wait()
            for i in range(CW // 16):
                scur[b][pl.ds(i * 16, 16)] = sring[kk, pl.ds(i * 16, 16)]
                dcur[b][pl.ds(i * 16, 16)] = dring[kk, pl.ds(i * 16, 16)]
            gds[b] = pltpu.async_copy(table.at[scur[b]], gbuf[b], gsem[b])
            if kk >= 1:
                b2 = (kk - 1) % NBUF
                gds[b2].wait()
                sds[b2] = pltpu.async_copy(gbuf[b2], acc.at[dcur[b2]], ssem[b2], add=True)
        bl = (GRP - 1) % NBUF
        gds[bl].wait()
        sds[bl] = pltpu.async_copy(gbuf[bl], acc.at[dcur[bl]], ssem[bl], add=True)
        for b in range(NBUF):
            sds[b].wait()
        return carry

    lax.fori_loop(0, CH // GRP, group, 0)
    plsc.subcore_barrier()
    pltpu.sync_copy(acc.at[pl.ds(s * RPT, RPT)], out_hbm.at[c, pl.ds(s * RPT, RPT)])


def _dinv_block(degp):
    return lax.rsqrt(degp[0, :, 0:1])


def _pre_body(x_ref, w_ref, degp_ref, out_ref):
    h = jnp.dot(x_ref[...], w_ref[...], preferred_element_type=jnp.float32)
    hs = h * _dinv_block(degp_ref[...])
    out_ref[0, :, :] = hs[:, :HH]
    out_ref[1, :, :] = hs[:, HH:]


def _tc_pre(x, W1, degp):
    return pl.pallas_call(
        _pre_body,
        grid=(NBLK,),
        in_specs=[
            pl.BlockSpec((RB, D), lambda i: (i, 0)),
            pl.BlockSpec((D, H), lambda i: (0, 0)),
            pl.BlockSpec((2, RB, HH), lambda i: (0, i, 0)),
        ],
        out_specs=pl.BlockSpec((2, RB, HH), lambda i: (0, i, 0)),
        out_shape=jax.ShapeDtypeStruct((2, NP, HH), jnp.float32),
    )(x, W1, degp)


def _z_block(agg_ref, degp_ref, b_ref):
    agg = jnp.concatenate([agg_ref[0], agg_ref[1]], axis=1)
    return agg * _dinv_block(degp_ref[...]) + b_ref[...]


def _gn_coeffs(s1_ref, s2_ref, w_ref, b_ref, a_ref, eps=1e-5):
    m = s1_ref[...] / N
    var = s2_ref[...] / N - m * m * a_ref[...] * (2.0 - a_ref[...])
    cmul = w_ref[...] * lax.rsqrt(var + eps)
    cadd = b_ref[...] - a_ref[...] * m * cmul
    return cmul, cadd


def _mid_body(agg_ref, degp_ref, b1_ref, w_ref, bb_ref, a_ref, w2_ref,
              out_ref, s1, s2, cmul, cadd):
    j = pl.program_id(0)
    i = pl.program_id(1)

    @pl.when(jnp.logical_and(j == 0, i == 0))
    def _():
        s1[...] = jnp.zeros_like(s1)
        s2[...] = jnp.zeros_like(s2)

    z = _z_block(agg_ref, degp_ref, b1_ref)

    @pl.when(j == 0)
    def _():
        s1[...] += jnp.sum(z, axis=0, keepdims=True)
        s2[...] += jnp.sum(z * z, axis=0, keepdims=True)

    @pl.when(jnp.logical_and(j == 1, i == 0))
    def _():
        cm, ca = _gn_coeffs(s1, s2, w_ref, bb_ref, a_ref)
        cmul[...] = cm
        cadd[...] = ca

    @pl.when(j == 1)
    def _():
        g = jnp.maximum(z * cmul[...] + cadd[...], 0.0)
        h2 = jnp.dot(g, w2_ref[...], preferred_element_type=jnp.float32)
        hs = h2 * _dinv_block(degp_ref[...])
        out_ref[0, :, :] = hs[:, :HH]
        out_ref[1, :, :] = hs[:, HH:]


def _tc_mid(agg1, degp, b1, gn1_w, gn1_b, gn1_a, W2):
    return pl.pallas_call(
        _mid_body,
        grid=(2, NBLK),
        in_specs=[
            pl.BlockSpec((2, RB, HH), lambda j, i: (0, i, 0)),
            pl.BlockSpec((2, RB, HH), lambda j, i: (0, i, 0)),
            pl.BlockSpec((1, H), lambda j, i: (0, 0)),
            pl.BlockSpec((1, H), lambda j, i: (0, 0)),
            pl.BlockSpec((1, H), lambda j, i: (0, 0)),
            pl.BlockSpec((1, H), lambda j, i: (0, 0)),
            pl.BlockSpec((H, H), lambda j, i: (0, 0)),
        ],
        out_specs=pl.BlockSpec((2, RB, HH), lambda j, i: (0, i, 0)),
        out_shape=jax.ShapeDtypeStruct((2, NP, HH), jnp.float32),
        scratch_shapes=[
            pltpu.VMEM((1, H), jnp.float32),
            pltpu.VMEM((1, H), jnp.float32),
            pltpu.VMEM((1, H), jnp.float32),
            pltpu.VMEM((1, H), jnp.float32),
        ],
        compiler_params=pltpu.CompilerParams(
            dimension_semantics=("arbitrary", "arbitrary")),
    )(agg1, degp, b1.reshape(1, H), gn1_w.reshape(1, H),
      gn1_b.reshape(1, H), gn1_a.reshape(1, H), W2)


def _fin_body(agg_ref, degp_ref, b2_ref, w_ref, bb_ref, a_ref, x_ref,
              out_ref, s1, s2, cmul, cadd):
    j = pl.program_id(0)
    i = pl.program_id(1)

    @pl.when(jnp.logical_and(j == 0, i == 0))
    def _():
        s1[...] = jnp.zeros_like(s1)
        s2[...] = jnp.zeros_like(s2)

    z = _z_block(agg_ref, degp_ref, b2_ref)

    @pl.when(j == 0)
    def _():
        s1[...] += jnp.sum(z, axis=0, keepdims=True)
        s2[...] += jnp.sum(z * z, axis=0, keepdims=True)

    @pl.when(jnp.logical_and(j == 1, i == 0))
    def _():
        cm, ca = _gn_coeffs(s1, s2, w_ref, bb_ref, a_ref)
        cmul[...] = cm
        cadd[...] = ca

    @pl.when(j == 1)
    def _():
        out_ref[...] = jnp.maximum(z * cmul[...] + cadd[...] + x_ref[...], 0.0)


def _tc_fin(agg2, degp, b2, gn2_w, gn2_b, gn2_a, x):
    return pl.pallas_call(
        _fin_body,
        grid=(2, NBLK),
        in_specs=[
            pl.BlockSpec((2, RB, HH), lambda j, i: (0, i, 0)),
            pl.BlockSpec((2, RB, HH), lambda j, i: (0, i, 0)),
            pl.BlockSpec((1, H), lambda j, i: (0, 0)),
            pl.BlockSpec((1, H), lambda j, i: (0, 0)),
            pl.BlockSpec((1, H), lambda j, i: (0, 0)),
            pl.BlockSpec((1, H), lambda j, i: (0, 0)),
            pl.BlockSpec((RB, H), lambda j, i: (i, 0)),
        ],
        out_specs=pl.BlockSpec((RB, H), lambda j, i: (i, 0)),
        out_shape=jax.ShapeDtypeStruct((N, H), jnp.float32),
        scratch_shapes=[
            pltpu.VMEM((1, H), jnp.float32),
            pltpu.VMEM((1, H), jnp.float32),
            pltpu.VMEM((1, H), jnp.float32),
            pltpu.VMEM((1, H), jnp.float32),
        ],
        compiler_params=pltpu.CompilerParams(
            dimension_semantics=("arbitrary", "arbitrary")),
    )(agg2, degp, b2.reshape(1, H), gn2_w.reshape(1, H),
      gn2_b.reshape(1, H), gn2_a.reshape(1, H), x)


def _conv_jnp(hs, src, dst):
    return hs.at[:, dst, :].add(hs[:, src, :])


def kernel(x, edge_index, W1, b1, W2, b2, gn1_w, gn1_b, gn1_a, gn2_w, gn2_b, gn2_a):
    pad = jnp.full((2, EP - E), N, dtype=edge_index.dtype)
    ei = jnp.concatenate([edge_index, pad], axis=1)
    src = ei[0].reshape(NS, CH, CW)
    dst = ei[1].reshape(NS, CH, CW)
    ones_t = jnp.ones((NC, NP, HH), jnp.float32)

    degp = _conv_kernel(ones_t, src, dst)
    hs1 = _tc_pre(x, W1, degp)
    agg1 = _conv_kernel(hs1, src, dst)
    hs2 = _tc_mid(agg1, degp, b1, gn1_w, gn1_b, gn1_a, W2)
    agg2 = _conv_kernel(hs2, src, dst)
    return _tc_fin(agg2, degp, b2, gn2_w, gn2_b, gn2_a, x)


# speedup vs baseline: 1.3619x; 1.3619x over previous
import functools

import jax
import jax.numpy as jnp
from jax import lax
from jax.experimental import pallas as pl
from jax.experimental.pallas import tpu as pltpu
from jax.experimental.pallas import tpu_sc as plsc

N = 10000
NP = 10240
E = 160000
EP = 163840
D = 256
H = 256
HH = H // 2
NC, NS = 2, 16
CW = 128
EPT = EP // NS
CH = EPT // CW
CHD = EPT // NC // CW
RPT = NP // NS
RB = 1000
NBLK = N // RB


def _sc_mesh():
    return plsc.VectorSubcoreMesh(core_axis_name="c", subcore_axis_name="s")


CHD = (EP // (NC * NS)) // CW


@functools.partial(
    pl.kernel,
    out_type=jax.ShapeDtypeStruct((NC, NP, 128), jnp.float32),
    mesh=_sc_mesh(),
    scratch_types=[
        pltpu.VMEM((CHD, CW), jnp.int32),
        pltpu.VMEM((CW,), jnp.int32),
        pltpu.VMEM((CW, 128), jnp.float32),
        pltpu.VMEM_SHARED((NP, 128), jnp.float32),
    ],
)
def _deg_kernel(dst_hbm, zeros_hbm, ones_hbm, out_hbm, dst_v, dst_cur, ones_v, acc):
    c = lax.axis_index("c")
    s = lax.axis_index("s")
    pltpu.sync_copy(dst_hbm.at[s, pl.ds(c * CHD, CHD)], dst_v)
    pltpu.sync_copy(ones_hbm, ones_v)
    pltpu.sync_copy(zeros_hbm.at[pl.ds(s * RPT, RPT)], acc.at[pl.ds(s * RPT, RPT)])
    plsc.subcore_barrier()

    def body(k, carry):
        for i in range(CW // 16):
            dst_cur[pl.ds(i * 16, 16)] = dst_v[k, pl.ds(i * 16, 16)]
        pltpu.sync_copy(ones_v, acc.at[dst_cur], add=True)
        return carry

    lax.fori_loop(0, CHD, body, 0)
    plsc.subcore_barrier()
    pltpu.sync_copy(acc.at[pl.ds(s * RPT, RPT)], out_hbm.at[c, pl.ds(s * RPT, RPT)])


NBUF = 2
GRP = 8


@functools.partial(
    pl.kernel,
    out_type=jax.ShapeDtypeStruct((NC, NP, HH), jnp.float32),
    mesh=_sc_mesh(),
    scratch_types=(
        [pltpu.VMEM((GRP, CW), jnp.int32)] * 2
        + [pltpu.VMEM((CW,), jnp.int32)] * (2 * NBUF)
        + [pltpu.VMEM((CW, HH), jnp.float32)] * NBUF
        + [pltpu.VMEM_SHARED((NP, HH), jnp.float32)]
        + [pltpu.SemaphoreType.DMA] * (2 * NBUF)
    ),
)
def _conv_kernel(hs_hbm, src_hbm, dst_hbm, out_hbm, sring, dring, *rest):
    scur = rest[0:NBUF]
    dcur = rest[NBUF:2 * NBUF]
    gbuf = rest[2 * NBUF:3 * NBUF]
    acc = rest[3 * NBUF]
    gsem = rest[3 * NBUF + 1:3 * NBUF + 1 + NBUF]
    ssem = rest[3 * NBUF + 1 + NBUF:3 * NBUF + 1 + 2 * NBUF]
    c = lax.axis_index("c")
    s = lax.axis_index("s")
    pltpu.sync_copy(hs_hbm.at[c, pl.ds(s * RPT, RPT)], acc.at[pl.ds(s * RPT, RPT)])
    plsc.subcore_barrier()
    table = hs_hbm.at[c]

    def group(gg, carry):
        pltpu.sync_copy(src_hbm.at[s, pl.ds(gg * GRP, GRP)], sring)
        pltpu.sync_copy(dst_hbm.at[s, pl.ds(gg * GRP, GRP)], dring)
        gds = [None] * NBUF
        sds = [None] * NBUF
        for kk in range(GRP):
            b = kk % NBUF
            if sds[b] is not None:
                sds[b].wait()
            for i in range(CW // 16):
                scur[b][pl.ds(i * 16, 16)] = sring[kk, pl.ds(i * 16, 16)]
                dcur[b][pl.ds(i * 16, 16)] = dring[kk, pl.ds(i * 16, 16)]
            gds[b] = pltpu.async_copy(table.at[scur[b]], gbuf[b], gsem[b])
            if kk >= 1:
                b2 = (kk - 1) % NBUF
                gds[b2].wait()
                sds[b2] = pltpu.async_copy(gbuf[b2], acc.at[dcur[b2]], ssem[b2], add=True)
        bl = (GRP - 1) % NBUF
        gds[bl].wait()
        sds[bl] = pltpu.async_copy(gbuf[bl], acc.at[dcur[bl]], ssem[bl], add=True)
        for b in range(NBUF):
            sds[b].wait()
        return carry

    lax.fori_loop(0, CH // GRP, group, 0)
    plsc.subcore_barrier()
    pltpu.sync_copy(acc.at[pl.ds(s * RPT, RPT)], out_hbm.at[c, pl.ds(s * RPT, RPT)])


def _dinv_block(degp):
    deg = degp[0, :, 0:1] + degp[1, :, 0:1] + 1.0
    return lax.rsqrt(deg)


def _pre_body(x_ref, w_ref, degp_ref, out_ref):
    h = jnp.dot(x_ref[...], w_ref[...], preferred_element_type=jnp.float32)
    hs = h * _dinv_block(degp_ref[...])
    out_ref[0, :, :] = hs[:, :HH]
    out_ref[1, :, :] = hs[:, HH:]


def _tc_pre(x, W1, degp):
    return pl.pallas_call(
        _pre_body,
        grid=(NBLK,),
        in_specs=[
            pl.BlockSpec((RB, D), lambda i: (i, 0)),
            pl.BlockSpec((D, H), lambda i: (0, 0)),
            pl.BlockSpec((2, RB, HH), lambda i: (0, i, 0)),
        ],
        out_specs=pl.BlockSpec((2, RB, HH), lambda i: (0, i, 0)),
        out_shape=jax.ShapeDtypeStruct((2, NP, HH), jnp.float32),
    )(x, W1, degp)


def _z_block(agg_ref, degp_ref, b_ref):
    agg = jnp.concatenate([agg_ref[0], agg_ref[1]], axis=1)
    return agg * _dinv_block(degp_ref[...]) + b_ref[...]


def _gn_coeffs(s1_ref, s2_ref, w_ref, b_ref, a_ref, eps=1e-5):
    m = s1_ref[...] / N
    var = s2_ref[...] / N - m * m * a_ref[...] * (2.0 - a_ref[...])
    cmul = w_ref[...] * lax.rsqrt(var + eps)
    cadd = b_ref[...] - a_ref[...] * m * cmul
    return cmul, cadd


def _mid_body(agg_ref, degp_ref, b1_ref, w_ref, bb_ref, a_ref, w2_ref,
              out_ref, s1, s2, cmul, cadd):
    j = pl.program_id(0)
    i = pl.program_id(1)

    @pl.when(jnp.logical_and(j == 0, i == 0))
    def _():
        s1[...] = jnp.zeros_like(s1)
        s2[...] = jnp.zeros_like(s2)

    z = _z_block(agg_ref, degp_ref, b1_ref)

    @pl.when(j == 0)
    def _():
        s1[...] += jnp.sum(z, axis=0, keepdims=True)
        s2[...] += jnp.sum(z * z, axis=0, keepdims=True)

    @pl.when(jnp.logical_and(j == 1, i == 0))
    def _():
        cm, ca = _gn_coeffs(s1, s2, w_ref, bb_ref, a_ref)
        cmul[...] = cm
        cadd[...] = ca

    @pl.when(j == 1)
    def _():
        g = jnp.maximum(z * cmul[...] + cadd[...], 0.0)
        h2 = jnp.dot(g, w2_ref[...], preferred_element_type=jnp.float32)
        hs = h2 * _dinv_block(degp_ref[...])
        out_ref[0, :, :] = hs[:, :HH]
        out_ref[1, :, :] = hs[:, HH:]


def _tc_mid(agg1, degp, b1, gn1_w, gn1_b, gn1_a, W2):
    return pl.pallas_call(
        _mid_body,
        grid=(2, NBLK),
        in_specs=[
            pl.BlockSpec((2, RB, HH), lambda j, i: (0, i, 0)),
            pl.BlockSpec((2, RB, HH), lambda j, i: (0, i, 0)),
            pl.BlockSpec((1, H), lambda j, i: (0, 0)),
            pl.BlockSpec((1, H), lambda j, i: (0, 0)),
            pl.BlockSpec((1, H), lambda j, i: (0, 0)),
            pl.BlockSpec((1, H), lambda j, i: (0, 0)),
            pl.BlockSpec((H, H), lambda j, i: (0, 0)),
        ],
        out_specs=pl.BlockSpec((2, RB, HH), lambda j, i: (0, i, 0)),
        out_shape=jax.ShapeDtypeStruct((2, NP, HH), jnp.float32),
        scratch_shapes=[
            pltpu.VMEM((1, H), jnp.float32),
            pltpu.VMEM((1, H), jnp.float32),
            pltpu.VMEM((1, H), jnp.float32),
            pltpu.VMEM((1, H), jnp.float32),
        ],
        compiler_params=pltpu.CompilerParams(
            dimension_semantics=("arbitrary", "arbitrary")),
    )(agg1, degp, b1.reshape(1, H), gn1_w.reshape(1, H),
      gn1_b.reshape(1, H), gn1_a.reshape(1, H), W2)


def _fin_body(agg_ref, degp_ref, b2_ref, w_ref, bb_ref, a_ref, x_ref,
              out_ref, s1, s2, cmul, cadd):
    j = pl.program_id(0)
    i = pl.program_id(1)

    @pl.when(jnp.logical_and(j == 0, i == 0))
    def _():
        s1[...] = jnp.zeros_like(s1)
        s2[...] = jnp.zeros_like(s2)

    z = _z_block(agg_ref, degp_ref, b2_ref)

    @pl.when(j == 0)
    def _():
        s1[...] += jnp.sum(z, axis=0, keepdims=True)
        s2[...] += jnp.sum(z * z, axis=0, keepdims=True)

    @pl.when(jnp.logical_and(j == 1, i == 0))
    def _():
        cm, ca = _gn_coeffs(s1, s2, w_ref, bb_ref, a_ref)
        cmul[...] = cm
        cadd[...] = ca

    @pl.when(j == 1)
    def _():
        out_ref[...] = jnp.maximum(z * cmul[...] + cadd[...] + x_ref[...], 0.0)


def _tc_fin(agg2, degp, b2, gn2_w, gn2_b, gn2_a, x):
    return pl.pallas_call(
        _fin_body,
        grid=(2, NBLK),
        in_specs=[
            pl.BlockSpec((2, RB, HH), lambda j, i: (0, i, 0)),
            pl.BlockSpec((2, RB, HH), lambda j, i: (0, i, 0)),
            pl.BlockSpec((1, H), lambda j, i: (0, 0)),
            pl.BlockSpec((1, H), lambda j, i: (0, 0)),
            pl.BlockSpec((1, H), lambda j, i: (0, 0)),
            pl.BlockSpec((1, H), lambda j, i: (0, 0)),
            pl.BlockSpec((RB, H), lambda j, i: (i, 0)),
        ],
        out_specs=pl.BlockSpec((RB, H), lambda j, i: (i, 0)),
        out_shape=jax.ShapeDtypeStruct((N, H), jnp.float32),
        scratch_shapes=[
            pltpu.VMEM((1, H), jnp.float32),
            pltpu.VMEM((1, H), jnp.float32),
            pltpu.VMEM((1, H), jnp.float32),
            pltpu.VMEM((1, H), jnp.float32),
        ],
        compiler_params=pltpu.CompilerParams(
            dimension_semantics=("arbitrary", "arbitrary")),
    )(agg2, degp, b2.reshape(1, H), gn2_w.reshape(1, H),
      gn2_b.reshape(1, H), gn2_a.reshape(1, H), x)


def _conv_jnp(hs, src, dst):
    return hs.at[:, dst, :].add(hs[:, src, :])


def kernel(x, edge_index, W1, b1, W2, b2, gn1_w, gn1_b, gn1_a, gn2_w, gn2_b, gn2_a):
    pad = jnp.full((2, EP - E), N, dtype=edge_index.dtype)
    ei = jnp.concatenate([edge_index, pad], axis=1)
    src = ei[0].reshape(NS, CH, CW)
    dst = ei[1].reshape(NS, CH, CW)
    zeros128 = jnp.zeros((NP, 128), jnp.float32)
    ones128 = jnp.ones((CW, 128), jnp.float32)

    degp = _deg_kernel(dst, zeros128, ones128)
    hs1 = _tc_pre(x, W1, degp)
    agg1 = _conv_kernel(hs1, src, dst)
    hs2 = _tc_mid(agg1, degp, b1, gn1_w, gn1_b, gn1_a, W2)
    agg2 = _conv_kernel(hs2, src, dst)
    return _tc_fin(agg2, degp, b2, gn2_w, gn2_b, gn2_a, x)


# speedup vs baseline: 1.3840x; 1.0162x over previous
import functools

import jax
import jax.numpy as jnp
from jax import lax
from jax.experimental import pallas as pl
from jax.experimental.pallas import tpu as pltpu
from jax.experimental.pallas import tpu_sc as plsc

N = 10000
NP = 10240
E = 160000
EP = 163840
D = 256
H = 256
HH = H // 2
NC, NS = 2, 16
CW = 128
EPT = EP // NS
CH = EPT // CW
CHD = EPT // NC // CW
RPT = NP // NS
RB = 1000
NBLK = N // RB


def _sc_mesh():
    return plsc.VectorSubcoreMesh(core_axis_name="c", subcore_axis_name="s")


CHD = (EP // (NC * NS)) // CW


@functools.partial(
    pl.kernel,
    out_type=jax.ShapeDtypeStruct((NC, NP, 128), jnp.float32),
    mesh=_sc_mesh(),
    scratch_types=[
        pltpu.VMEM((CHD, CW), jnp.int32),
        pltpu.VMEM((CW,), jnp.int32),
        pltpu.VMEM((CW, 128), jnp.float32),
        pltpu.VMEM_SHARED((NP, 128), jnp.float32),
    ],
)
def _deg_kernel(dst_hbm, zeros_hbm, ones_hbm, out_hbm, dst_v, dst_cur, ones_v, acc):
    c = lax.axis_index("c")
    s = lax.axis_index("s")
    pltpu.sync_copy(dst_hbm.at[s, pl.ds(c * CHD, CHD)], dst_v)
    pltpu.sync_copy(ones_hbm, ones_v)
    pltpu.sync_copy(zeros_hbm.at[pl.ds(s * RPT, RPT)], acc.at[pl.ds(s * RPT, RPT)])
    plsc.subcore_barrier()

    def body(k, carry):
        for i in range(CW // 16):
            dst_cur[pl.ds(i * 16, 16)] = dst_v[k, pl.ds(i * 16, 16)]
        pltpu.sync_copy(ones_v, acc.at[dst_cur], add=True)
        return carry

    lax.fori_loop(0, CHD, body, 0)
    plsc.subcore_barrier()
    pltpu.sync_copy(acc.at[pl.ds(s * RPT, RPT)], out_hbm.at[c, pl.ds(s * RPT, RPT)])


NB = 4
SUB = 64
LAG = 2
GRP = 8
SG = GRP * CW // SUB


@functools.partial(
    pl.kernel,
    out_type=jax.ShapeDtypeStruct((NC, NP, HH), jnp.float32),
    mesh=_sc_mesh(),
    scratch_types=(
        [pltpu.VMEM((GRP, CW), jnp.int32)] * 2
        + [pltpu.VMEM((SUB,), jnp.int32)] * (2 * NB)
        + [pltpu.VMEM((SUB, HH), jnp.float32)] * NB
        + [pltpu.VMEM_SHARED((NP, HH), jnp.float32)]
        + [pltpu.SemaphoreType.DMA] * (2 * NB)
    ),
)
def _conv_kernel(hs_hbm, src_hbm, dst_hbm, out_hbm, sring, dring, *rest):
    scur = rest[0:NB]
    dcur = rest[NB:2 * NB]
    gbuf = rest[2 * NB:3 * NB]
    acc = rest[3 * NB]
    gsem = rest[3 * NB + 1:3 * NB + 1 + NB]
    ssem = rest[3 * NB + 1 + NB:3 * NB + 1 + 2 * NB]
    c = lax.axis_index("c")
    s = lax.axis_index("s")
    pltpu.sync_copy(hs_hbm.at[c, pl.ds(s * RPT, RPT)], acc.at[pl.ds(s * RPT, RPT)])
    plsc.subcore_barrier()
    table = hs_hbm.at[c]

    def fire_scatter(gds, sds, q):
        b2 = q % NB
        gds[b2].wait()
        sds[b2] = pltpu.async_copy(gbuf[b2], acc.at[dcur[b2]], ssem[b2], add=True)

    def group(gg, carry):
        pltpu.sync_copy(src_hbm.at[s, pl.ds(gg * GRP, GRP)], sring)
        pltpu.sync_copy(dst_hbm.at[s, pl.ds(gg * GRP, GRP)], dring)
        gds = [None] * NB
        sds = [None] * NB
        for q in range(SG):
            b = q % NB
            if sds[b] is not None:
                sds[b].wait()
            kk, off = q // (CW // SUB), (q % (CW // SUB)) * SUB
            for i in range(SUB // 16):
                scur[b][pl.ds(i * 16, 16)] = sring[kk, pl.ds(off + i * 16, 16)]
                dcur[b][pl.ds(i * 16, 16)] = dring[kk, pl.ds(off + i * 16, 16)]
            gds[b] = pltpu.async_copy(table.at[scur[b]], gbuf[b], gsem[b])
            if q >= LAG:
                fire_scatter(gds, sds, q - LAG)
        for q in range(SG - LAG, SG):
            fire_scatter(gds, sds, q)
        for b in range(NB):
            sds[b].wait()
        return carry

    lax.fori_loop(0, CH // GRP, group, 0)
    plsc.subcore_barrier()
    pltpu.sync_copy(acc.at[pl.ds(s * RPT, RPT)], out_hbm.at[c, pl.ds(s * RPT, RPT)])


def _dinv_block(degp):
    deg = degp[0, :, 0:1] + degp[1, :, 0:1] + 1.0
    return lax.rsqrt(deg)


def _pre_body(x_ref, w_ref, degp_ref, out_ref):
    h = jnp.dot(x_ref[...], w_ref[...], preferred_element_type=jnp.float32)
    hs = h * _dinv_block(degp_ref[...])
    out_ref[0, :, :] = hs[:, :HH]
    out_ref[1, :, :] = hs[:, HH:]


def _tc_pre(x, W1, degp):
    return pl.pallas_call(
        _pre_body,
        grid=(NBLK,),
        in_specs=[
            pl.BlockSpec((RB, D), lambda i: (i, 0)),
            pl.BlockSpec((D, H), lambda i: (0, 0)),
            pl.BlockSpec((2, RB, HH), lambda i: (0, i, 0)),
        ],
        out_specs=pl.BlockSpec((2, RB, HH), lambda i: (0, i, 0)),
        out_shape=jax.ShapeDtypeStruct((2, NP, HH), jnp.float32),
    )(x, W1, degp)


def _z_block(agg_ref, degp_ref, b_ref):
    agg = jnp.concatenate([agg_ref[0], agg_ref[1]], axis=1)
    return agg * _dinv_block(degp_ref[...]) + b_ref[...]


def _gn_coeffs(s1_ref, s2_ref, w_ref, b_ref, a_ref, eps=1e-5):
    m = s1_ref[...] / N
    var = s2_ref[...] / N - m * m * a_ref[...] * (2.0 - a_ref[...])
    cmul = w_ref[...] * lax.rsqrt(var + eps)
    cadd = b_ref[...] - a_ref[...] * m * cmul
    return cmul, cadd


def _mid_body(agg_ref, degp_ref, b1_ref, w_ref, bb_ref, a_ref, w2_ref,
              out_ref, s1, s2, cmul, cadd):
    j = pl.program_id(0)
    i = pl.program_id(1)

    @pl.when(jnp.logical_and(j == 0, i == 0))
    def _():
        s1[...] = jnp.zeros_like(s1)
        s2[...] = jnp.zeros_like(s2)

    z = _z_block(agg_ref, degp_ref, b1_ref)

    @pl.when(j == 0)
    def _():
        s1[...] += jnp.sum(z, axis=0, keepdims=True)
        s2[...] += jnp.sum(z * z, axis=0, keepdims=True)

    @pl.when(jnp.logical_and(j == 1, i == 0))
    def _():
        cm, ca = _gn_coeffs(s1, s2, w_ref, bb_ref, a_ref)
        cmul[...] = cm
        cadd[...] = ca

    @pl.when(j == 1)
    def _():
        g = jnp.maximum(z * cmul[...] + cadd[...], 0.0)
        h2 = jnp.dot(g, w2_ref[...], preferred_element_type=jnp.float32)
        hs = h2 * _dinv_block(degp_ref[...])
        out_ref[0, :, :] = hs[:, :HH]
        out_ref[1, :, :] = hs[:, HH:]


def _tc_mid(agg1, degp, b1, gn1_w, gn1_b, gn1_a, W2):
    return pl.pallas_call(
        _mid_body,
        grid=(2, NBLK),
        in_specs=[
            pl.BlockSpec((2, RB, HH), lambda j, i: (0, i, 0)),
            pl.BlockSpec((2, RB, HH), lambda j, i: (0, i, 0)),
            pl.BlockSpec((1, H), lambda j, i: (0, 0)),
            pl.BlockSpec((1, H), lambda j, i: (0, 0)),
            pl.BlockSpec((1, H), lambda j, i: (0, 0)),
            pl.BlockSpec((1, H), lambda j, i: (0, 0)),
            pl.BlockSpec((H, H), lambda j, i: (0, 0)),
        ],
        out_specs=pl.BlockSpec((2, RB, HH), lambda j, i: (0, i, 0)),
        out_shape=jax.ShapeDtypeStruct((2, NP, HH), jnp.float32),
        scratch_shapes=[
            pltpu.VMEM((1, H), jnp.float32),
            pltpu.VMEM((1, H), jnp.float32),
            pltpu.VMEM((1, H), jnp.float32),
            pltpu.VMEM((1, H), jnp.float32),
        ],
        compiler_params=pltpu.CompilerParams(
            dimension_semantics=("arbitrary", "arbitrary")),
    )(agg1, degp, b1.reshape(1, H), gn1_w.reshape(1, H),
      gn1_b.reshape(1, H), gn1_a.reshape(1, H), W2)


def _fin_body(agg_ref, degp_ref, b2_ref, w_ref, bb_ref, a_ref, x_ref,
              out_ref, s1, s2, cmul, cadd):
    j = pl.program_id(0)
    i = pl.program_id(1)

    @pl.when(jnp.logical_and(j == 0, i == 0))
    def _():
        s1[...] = jnp.zeros_like(s1)
        s2[...] = jnp.zeros_like(s2)

    z = _z_block(agg_ref, degp_ref, b2_ref)

    @pl.when(j == 0)
    def _():
        s1[...] += jnp.sum(z, axis=0, keepdims=True)
        s2[...] += jnp.sum(z * z, axis=0, keepdims=True)

    @pl.when(jnp.logical_and(j == 1, i == 0))
    def _():
        cm, ca = _gn_coeffs(s1, s2, w_ref, bb_ref, a_ref)
        cmul[...] = cm
        cadd[...] = ca

    @pl.when(j == 1)
    def _():
        out_ref[...] = jnp.maximum(z * cmul[...] + cadd[...] + x_ref[...], 0.0)


def _tc_fin(agg2, degp, b2, gn2_w, gn2_b, gn2_a, x):
    return pl.pallas_call(
        _fin_body,
        grid=(2, NBLK),
        in_specs=[
            pl.BlockSpec((2, RB, HH), lambda j, i: (0, i, 0)),
            pl.BlockSpec((2, RB, HH), lambda j, i: (0, i, 0)),
            pl.BlockSpec((1, H), lambda j, i: (0, 0)),
            pl.BlockSpec((1, H), lambda j, i: (0, 0)),
            pl.BlockSpec((1, H), lambda j, i: (0, 0)),
            pl.BlockSpec((1, H), lambda j, i: (0, 0)),
            pl.BlockSpec((RB, H), lambda j, i: (i, 0)),
        ],
        out_specs=pl.BlockSpec((RB, H), lambda j, i: (i, 0)),
        out_shape=jax.ShapeDtypeStruct((N, H), jnp.float32),
        scratch_shapes=[
            pltpu.VMEM((1, H), jnp.float32),
            pltpu.VMEM((1, H), jnp.float32),
            pltpu.VMEM((1, H), jnp.float32),
            pltpu.VMEM((1, H), jnp.float32),
        ],
        compiler_params=pltpu.CompilerParams(
            dimension_semantics=("arbitrary", "arbitrary")),
    )(agg2, degp, b2.reshape(1, H), gn2_w.reshape(1, H),
      gn2_b.reshape(1, H), gn2_a.reshape(1, H), x)


def _conv_jnp(hs, src, dst):
    return hs.at[:, dst, :].add(hs[:, src, :])


def kernel(x, edge_index, W1, b1, W2, b2, gn1_w, gn1_b, gn1_a, gn2_w, gn2_b, gn2_a):
    pad = jnp.full((2, EP - E), N, dtype=edge_index.dtype)
    ei = jnp.concatenate([edge_index, pad], axis=1)
    src = ei[0].reshape(NS, CH, CW)
    dst = ei[1].reshape(NS, CH, CW)
    zeros128 = jnp.zeros((NP, 128), jnp.float32)
    ones128 = jnp.ones((CW, 128), jnp.float32)

    degp = _deg_kernel(dst, zeros128, ones128)
    hs1 = _tc_pre(x, W1, degp)
    agg1 = _conv_kernel(hs1, src, dst)
    hs2 = _tc_mid(agg1, degp, b1, gn1_w, gn1_b, gn1_a, W2)
    agg2 = _conv_kernel(hs2, src, dst)
    return _tc_fin(agg2, degp, b2, gn2_w, gn2_b, gn2_a, x)


# speedup vs baseline: 1.4338x; 1.0359x over previous
import functools

import jax
import jax.numpy as jnp
from jax import lax
from jax.experimental import pallas as pl
from jax.experimental.pallas import tpu as pltpu
from jax.experimental.pallas import tpu_sc as plsc

N = 10000
NP = 10240
E = 160000
EP = 163840
D = 256
H = 256
HH = H // 2
NC, NS = 2, 16
CW = 128
EPT = EP // NS
CH = EPT // CW
CHD = EPT // NC // CW
RPT = NP // NS
RB = 1000
NBLK = N // RB


def _sc_mesh():
    return plsc.VectorSubcoreMesh(core_axis_name="c", subcore_axis_name="s")


CHD = (EP // (NC * NS)) // CW


@functools.partial(
    pl.kernel,
    out_type=jax.ShapeDtypeStruct((NC, NP, 128), jnp.float32),
    mesh=_sc_mesh(),
    scratch_types=[
        pltpu.VMEM((CHD, CW), jnp.int32),
        pltpu.VMEM((CW,), jnp.int32),
        pltpu.VMEM((CW, 128), jnp.float32),
        pltpu.VMEM_SHARED((NP, 128), jnp.float32),
    ],
)
def _deg_kernel(dst_hbm, zeros_hbm, ones_hbm, out_hbm, dst_v, dst_cur, ones_v, acc):
    c = lax.axis_index("c")
    s = lax.axis_index("s")
    pltpu.sync_copy(dst_hbm.at[s, pl.ds(c * CHD, CHD)], dst_v)
    pltpu.sync_copy(ones_hbm, ones_v)
    pltpu.sync_copy(zeros_hbm.at[pl.ds(s * RPT, RPT)], acc.at[pl.ds(s * RPT, RPT)])
    plsc.subcore_barrier()

    def body(k, carry):
        for i in range(CW // 16):
            dst_cur[pl.ds(i * 16, 16)] = dst_v[k, pl.ds(i * 16, 16)]
        pltpu.sync_copy(ones_v, acc.at[dst_cur], add=True)
        return carry

    lax.fori_loop(0, CHD, body, 0)
    plsc.subcore_barrier()
    pltpu.sync_copy(acc.at[pl.ds(s * RPT, RPT)], out_hbm.at[c, pl.ds(s * RPT, RPT)])


NB = 4
SUB = 64
LAG = 2
GRP = 8
SG = GRP * CW // SUB


@functools.partial(
    pl.kernel,
    out_type=jax.ShapeDtypeStruct((NC, NP, HH), jnp.float32),
    mesh=_sc_mesh(),
    scratch_types=(
        [pltpu.VMEM((GRP, CW), jnp.int32)] * 2
        + [pltpu.VMEM((SUB,), jnp.int32)] * (2 * NB)
        + [pltpu.VMEM((SUB, HH), jnp.float32)] * NB
        + [pltpu.VMEM_SHARED((NP, HH), jnp.float32)]
        + [pltpu.SemaphoreType.DMA] * (2 * NB)
    ),
)
def _conv_kernel(hs_hbm, src_hbm, dst_hbm, out_hbm, sring, dring, *rest):
    scur = rest[0:NB]
    dcur = rest[NB:2 * NB]
    gbuf = rest[2 * NB:3 * NB]
    acc = rest[3 * NB]
    gsem = rest[3 * NB + 1:3 * NB + 1 + NB]
    ssem = rest[3 * NB + 1 + NB:3 * NB + 1 + 2 * NB]
    c = lax.axis_index("c")
    s = lax.axis_index("s")
    pltpu.sync_copy(hs_hbm.at[c, pl.ds(s * RPT, RPT)], acc.at[pl.ds(s * RPT, RPT)])
    plsc.subcore_barrier()
    table = hs_hbm.at[c]

    def fire_scatter(gds, sds, q):
        b2 = q % NB
        gds[b2].wait()
        sds[b2] = pltpu.async_copy(gbuf[b2], acc.at[dcur[b2]], ssem[b2], add=True)

    def group(gg, carry):
        pltpu.sync_copy(src_hbm.at[s, pl.ds(gg * GRP, GRP)], sring)
        pltpu.sync_copy(dst_hbm.at[s, pl.ds(gg * GRP, GRP)], dring)
        gds = [None] * NB
        sds = [None] * NB
        for q in range(SG):
            b = q % NB
            if sds[b] is not None:
                sds[b].wait()
            kk, off = q // (CW // SUB), (q % (CW // SUB)) * SUB
            for i in range(SUB // 16):
                scur[b][pl.ds(i * 16, 16)] = sring[kk, pl.ds(off + i * 16, 16)]
                dcur[b][pl.ds(i * 16, 16)] = dring[kk, pl.ds(off + i * 16, 16)]
            gds[b] = pltpu.async_copy(table.at[scur[b]], gbuf[b], gsem[b])
            if q >= LAG:
                fire_scatter(gds, sds, q - LAG)
        for q in range(SG - LAG, SG):
            fire_scatter(gds, sds, q)
        for b in range(NB):
            sds[b].wait()
        return carry

    lax.fori_loop(0, CH // GRP, group, 0)
    plsc.subcore_barrier()
    pltpu.sync_copy(acc.at[pl.ds(s * RPT, RPT)], out_hbm.at[c, pl.ds(s * RPT, RPT)])


def _dinv_block(degp):
    deg = degp[0, :, 0:1] + degp[1, :, 0:1] + 1.0
    return lax.rsqrt(deg)


def _pre_body(x_ref, w_ref, degp_ref, out_ref, dinv_ref):
    h = jnp.dot(x_ref[...], w_ref[...], preferred_element_type=jnp.float32)
    dinv = _dinv_block(degp_ref[...])
    hs = h * dinv
    out_ref[0, :, :] = hs[:, :HH]
    out_ref[1, :, :] = hs[:, HH:]
    dinv_ref[...] = dinv


def _tc_pre(x, W1, degp):
    return pl.pallas_call(
        _pre_body,
        grid=(NBLK,),
        in_specs=[
            pl.BlockSpec((RB, D), lambda i: (i, 0)),
            pl.BlockSpec((D, H), lambda i: (0, 0)),
            pl.BlockSpec((2, RB, HH), lambda i: (0, i, 0)),
        ],
        out_specs=[
            pl.BlockSpec((2, RB, HH), lambda i: (0, i, 0)),
            pl.BlockSpec((RB, 1), lambda i: (i, 0)),
        ],
        out_shape=[
            jax.ShapeDtypeStruct((2, NP, HH), jnp.float32),
            jax.ShapeDtypeStruct((N, 1), jnp.float32),
        ],
    )(x, W1, degp)


def _z_block(agg_ref, dinv_ref, b_ref):
    agg = jnp.concatenate([agg_ref[0], agg_ref[1]], axis=1)
    return agg * dinv_ref[...] + b_ref[...]


def _gn_coeffs(s1_ref, s2_ref, w_ref, b_ref, a_ref, eps=1e-5):
    m = s1_ref[...] / N
    var = s2_ref[...] / N - m * m * a_ref[...] * (2.0 - a_ref[...])
    cmul = w_ref[...] * lax.rsqrt(var + eps)
    cadd = b_ref[...] - a_ref[...] * m * cmul
    return cmul, cadd


def _mid_body(agg_ref, dinv_ref, b1_ref, w_ref, bb_ref, a_ref, w2_ref,
              out_ref, s1, s2, cmul, cadd):
    j = pl.program_id(0)
    i = pl.program_id(1)

    @pl.when(jnp.logical_and(j == 0, i == 0))
    def _():
        s1[...] = jnp.zeros_like(s1)
        s2[...] = jnp.zeros_like(s2)

    z = _z_block(agg_ref, dinv_ref, b1_ref)

    @pl.when(j == 0)
    def _():
        s1[...] += jnp.sum(z, axis=0, keepdims=True)
        s2[...] += jnp.sum(z * z, axis=0, keepdims=True)

    @pl.when(jnp.logical_and(j == 1, i == 0))
    def _():
        cm, ca = _gn_coeffs(s1, s2, w_ref, bb_ref, a_ref)
        cmul[...] = cm
        cadd[...] = ca

    @pl.when(j == 1)
    def _():
        g = jnp.maximum(z * cmul[...] + cadd[...], 0.0)
        h2 = jnp.dot(g, w2_ref[...], preferred_element_type=jnp.float32)
        hs = h2 * dinv_ref[...]
        out_ref[0, :, :] = hs[:, :HH]
        out_ref[1, :, :] = hs[:, HH:]


def _tc_mid(agg1, dinv, b1, gn1_w, gn1_b, gn1_a, W2):
    return pl.pallas_call(
        _mid_body,
        grid=(2, NBLK),
        in_specs=[
            pl.BlockSpec((2, RB, HH), lambda j, i: (0, i, 0)),
            pl.BlockSpec((RB, 1), lambda j, i: (i, 0)),
            pl.BlockSpec((1, H), lambda j, i: (0, 0)),
            pl.BlockSpec((1, H), lambda j, i: (0, 0)),
            pl.BlockSpec((1, H), lambda j, i: (0, 0)),
            pl.BlockSpec((1, H), lambda j, i: (0, 0)),
            pl.BlockSpec((H, H), lambda j, i: (0, 0)),
        ],
        out_specs=pl.BlockSpec((2, RB, HH), lambda j, i: (0, i, 0)),
        out_shape=jax.ShapeDtypeStruct((2, NP, HH), jnp.float32),
        scratch_shapes=[
            pltpu.VMEM((1, H), jnp.float32),
            pltpu.VMEM((1, H), jnp.float32),
            pltpu.VMEM((1, H), jnp.float32),
            pltpu.VMEM((1, H), jnp.float32),
        ],
        compiler_params=pltpu.CompilerParams(
            dimension_semantics=("arbitrary", "arbitrary")),
    )(agg1, dinv, b1.reshape(1, H), gn1_w.reshape(1, H),
      gn1_b.reshape(1, H), gn1_a.reshape(1, H), W2)


def _fin_body(agg_ref, dinv_ref, b2_ref, w_ref, bb_ref, a_ref, x_ref,
              out_ref, s1, s2, cmul, cadd):
    j = pl.program_id(0)
    i = pl.program_id(1)

    @pl.when(jnp.logical_and(j == 0, i == 0))
    def _():
        s1[...] = jnp.zeros_like(s1)
        s2[...] = jnp.zeros_like(s2)

    z = _z_block(agg_ref, dinv_ref, b2_ref)

    @pl.when(j == 0)
    def _():
        s1[...] += jnp.sum(z, axis=0, keepdims=True)
        s2[...] += jnp.sum(z * z, axis=0, keepdims=True)

    @pl.when(jnp.logical_and(j == 1, i == 0))
    def _():
        cm, ca = _gn_coeffs(s1, s2, w_ref, bb_ref, a_ref)
        cmul[...] = cm
        cadd[...] = ca

    @pl.when(j == 1)
    def _():
        out_ref[...] = jnp.maximum(z * cmul[...] + cadd[...] + x_ref[...], 0.0)


def _tc_fin(agg2, dinv, b2, gn2_w, gn2_b, gn2_a, x):
    return pl.pallas_call(
        _fin_body,
        grid=(2, NBLK),
        in_specs=[
            pl.BlockSpec((2, RB, HH), lambda j, i: (0, i, 0)),
            pl.BlockSpec((RB, 1), lambda j, i: (i, 0)),
            pl.BlockSpec((1, H), lambda j, i: (0, 0)),
            pl.BlockSpec((1, H), lambda j, i: (0, 0)),
            pl.BlockSpec((1, H), lambda j, i: (0, 0)),
            pl.BlockSpec((1, H), lambda j, i: (0, 0)),
            pl.BlockSpec((RB, H), lambda j, i: (i, 0)),
        ],
        out_specs=pl.BlockSpec((RB, H), lambda j, i: (i, 0)),
        out_shape=jax.ShapeDtypeStruct((N, H), jnp.float32),
        scratch_shapes=[
            pltpu.VMEM((1, H), jnp.float32),
            pltpu.VMEM((1, H), jnp.float32),
            pltpu.VMEM((1, H), jnp.float32),
            pltpu.VMEM((1, H), jnp.float32),
        ],
        compiler_params=pltpu.CompilerParams(
            dimension_semantics=("arbitrary", "arbitrary")),
    )(agg2, dinv, b2.reshape(1, H), gn2_w.reshape(1, H),
      gn2_b.reshape(1, H), gn2_a.reshape(1, H), x)


def _conv_jnp(hs, src, dst):
    return hs.at[:, dst, :].add(hs[:, src, :])


def kernel(x, edge_index, W1, b1, W2, b2, gn1_w, gn1_b, gn1_a, gn2_w, gn2_b, gn2_a):
    pad = jnp.full((2, EP - E), N, dtype=edge_index.dtype)
    ei = jnp.concatenate([edge_index, pad], axis=1)
    src = ei[0].reshape(NS, CH, CW)
    dst = ei[1].reshape(NS, CH, CW)
    zeros128 = jnp.zeros((NP, 128), jnp.float32)
    ones128 = jnp.ones((CW, 128), jnp.float32)

    degp = _deg_kernel(dst, zeros128, ones128)
    hs1, dinv = _tc_pre(x, W1, degp)
    agg1 = _conv_kernel(hs1, src, dst)
    hs2 = _tc_mid(agg1, dinv, b1, gn1_w, gn1_b, gn1_a, W2)
    agg2 = _conv_kernel(hs2, src, dst)
    return _tc_fin(agg2, dinv, b2, gn2_w, gn2_b, gn2_a, x)


# speedup vs baseline: 1.4360x; 1.0016x over previous
import functools

import jax
import jax.numpy as jnp
from jax import lax
from jax.experimental import pallas as pl
from jax.experimental.pallas import tpu as pltpu
from jax.experimental.pallas import tpu_sc as plsc

N = 10000
NP = 10240
E = 160000
EP = 163840
D = 256
H = 256
HH = H // 2
NC, NS = 2, 16
CW = 128
EPT = EP // NS
CH = EPT // CW
CHD = EPT // NC // CW
RPT = NP // NS
RB = 1000
NBLK = N // RB


def _sc_mesh():
    return plsc.VectorSubcoreMesh(core_axis_name="c", subcore_axis_name="s")


CHD = (EP // (NC * NS)) // CW


@functools.partial(
    pl.kernel,
    out_type=jax.ShapeDtypeStruct((NC, NP, 128), jnp.float32),
    mesh=_sc_mesh(),
    scratch_types=(
        [pltpu.VMEM((CHD, CW), jnp.int32)]
        + [pltpu.VMEM((CW,), jnp.int32)] * 4
        + [pltpu.VMEM((CW, 128), jnp.float32)]
        + [pltpu.VMEM_SHARED((NP, 128), jnp.float32)]
        + [pltpu.SemaphoreType.DMA] * 4
    ),
)
def _deg_kernel(dst_hbm, zeros_hbm, ones_hbm, out_hbm, dst_v, *rest):
    dcur = rest[0:4]
    ones_v = rest[4]
    acc = rest[5]
    sems = rest[6:10]
    c = lax.axis_index("c")
    s = lax.axis_index("s")
    pltpu.sync_copy(dst_hbm.at[s, pl.ds(c * CHD, CHD)], dst_v)
    pltpu.sync_copy(ones_hbm, ones_v)
    pltpu.sync_copy(zeros_hbm.at[pl.ds(s * RPT, RPT)], acc.at[pl.ds(s * RPT, RPT)])
    plsc.subcore_barrier()

    def body(g, carry):
        sds = []
        for b in range(4):
            k = g * 4 + b
            for i in range(CW // 16):
                dcur[b][pl.ds(i * 16, 16)] = dst_v[k, pl.ds(i * 16, 16)]
            sds.append(pltpu.async_copy(ones_v, acc.at[dcur[b]], sems[b], add=True))
        for b in range(4):
            sds[b].wait()
        return carry

    lax.fori_loop(0, CHD // 4, body, 0)
    plsc.subcore_barrier()
    pltpu.sync_copy(acc.at[pl.ds(s * RPT, RPT)], out_hbm.at[c, pl.ds(s * RPT, RPT)])


NB = 4
SUB = 64
LAG = 2
GRP = 8
SG = GRP * CW // SUB


@functools.partial(
    pl.kernel,
    out_type=jax.ShapeDtypeStruct((NC, NP, HH), jnp.float32),
    mesh=_sc_mesh(),
    scratch_types=(
        [pltpu.VMEM((GRP, CW), jnp.int32)] * 2
        + [pltpu.VMEM((SUB,), jnp.int32)] * (2 * NB)
        + [pltpu.VMEM((SUB, HH), jnp.float32)] * NB
        + [pltpu.VMEM_SHARED((NP, HH), jnp.float32)]
        + [pltpu.SemaphoreType.DMA] * (2 * NB)
    ),
)
def _conv_kernel(hs_hbm, src_hbm, dst_hbm, out_hbm, sring, dring, *rest):
    scur = rest[0:NB]
    dcur = rest[NB:2 * NB]
    gbuf = rest[2 * NB:3 * NB]
    acc = rest[3 * NB]
    gsem = rest[3 * NB + 1:3 * NB + 1 + NB]
    ssem = rest[3 * NB + 1 + NB:3 * NB + 1 + 2 * NB]
    c = lax.axis_index("c")
    s = lax.axis_index("s")
    pltpu.sync_copy(hs_hbm.at[c, pl.ds(s * RPT, RPT)], acc.at[pl.ds(s * RPT, RPT)])
    plsc.subcore_barrier()
    table = hs_hbm.at[c]

    def fire_scatter(gds, sds, q):
        b2 = q % NB
        gds[b2].wait()
        sds[b2] = pltpu.async_copy(gbuf[b2], acc.at[dcur[b2]], ssem[b2], add=True)

    def group(gg, carry):
        pltpu.sync_copy(src_hbm.at[s, pl.ds(gg * GRP, GRP)], sring)
        pltpu.sync_copy(dst_hbm.at[s, pl.ds(gg * GRP, GRP)], dring)
        gds = [None] * NB
        sds = [None] * NB
        for q in range(SG):
            b = q % NB
            if sds[b] is not None:
                sds[b].wait()
            kk, off = q // (CW // SUB), (q % (CW // SUB)) * SUB
            for i in range(SUB // 16):
                scur[b][pl.ds(i * 16, 16)] = sring[kk, pl.ds(off + i * 16, 16)]
                dcur[b][pl.ds(i * 16, 16)] = dring[kk, pl.ds(off + i * 16, 16)]
            gds[b] = pltpu.async_copy(table.at[scur[b]], gbuf[b], gsem[b])
            if q >= LAG:
                fire_scatter(gds, sds, q - LAG)
        for q in range(SG - LAG, SG):
            fire_scatter(gds, sds, q)
        for b in range(NB):
            sds[b].wait()
        return carry

    lax.fori_loop(0, CH // GRP, group, 0)
    plsc.subcore_barrier()
    pltpu.sync_copy(acc.at[pl.ds(s * RPT, RPT)], out_hbm.at[c, pl.ds(s * RPT, RPT)])


def _dinv_block(degp):
    deg = degp[0, :, 0:1] + degp[1, :, 0:1] + 1.0
    return lax.rsqrt(deg)


def _pre_body(x_ref, w_ref, degp_ref, out_ref, dinv_ref):
    h = jnp.dot(x_ref[...], w_ref[...], preferred_element_type=jnp.float32)
    dinv = _dinv_block(degp_ref[...])
    hs = h * dinv
    out_ref[0, :, :] = hs[:, :HH]
    out_ref[1, :, :] = hs[:, HH:]
    dinv_ref[...] = dinv


def _tc_pre(x, W1, degp):
    return pl.pallas_call(
        _pre_body,
        grid=(NBLK,),
        in_specs=[
            pl.BlockSpec((RB, D), lambda i: (i, 0)),
            pl.BlockSpec((D, H), lambda i: (0, 0)),
            pl.BlockSpec((2, RB, HH), lambda i: (0, i, 0)),
        ],
        out_specs=[
            pl.BlockSpec((2, RB, HH), lambda i: (0, i, 0)),
            pl.BlockSpec((RB, 1), lambda i: (i, 0)),
        ],
        out_shape=[
            jax.ShapeDtypeStruct((2, NP, HH), jnp.float32),
            jax.ShapeDtypeStruct((N, 1), jnp.float32),
        ],
    )(x, W1, degp)


def _z_block(agg_ref, dinv_ref, b_ref):
    agg = jnp.concatenate([agg_ref[0], agg_ref[1]], axis=1)
    return agg * dinv_ref[...] + b_ref[...]


def _gn_coeffs(s1_ref, s2_ref, w_ref, b_ref, a_ref, eps=1e-5):
    m = s1_ref[...] / N
    var = s2_ref[...] / N - m * m * a_ref[...] * (2.0 - a_ref[...])
    cmul = w_ref[...] * lax.rsqrt(var + eps)
    cadd = b_ref[...] - a_ref[...] * m * cmul
    return cmul, cadd


def _mid_body(agg_ref, dinv_ref, b1_ref, w_ref, bb_ref, a_ref, w2_ref,
              out_ref, s1, s2, cmul, cadd):
    j = pl.program_id(0)
    i = pl.program_id(1)

    @pl.when(jnp.logical_and(j == 0, i == 0))
    def _():
        s1[...] = jnp.zeros_like(s1)
        s2[...] = jnp.zeros_like(s2)

    z = _z_block(agg_ref, dinv_ref, b1_ref)

    @pl.when(j == 0)
    def _():
        s1[...] += jnp.sum(z, axis=0, keepdims=True)
        s2[...] += jnp.sum(z * z, axis=0, keepdims=True)

    @pl.when(jnp.logical_and(j == 1, i == 0))
    def _():
        cm, ca = _gn_coeffs(s1, s2, w_ref, bb_ref, a_ref)
        cmul[...] = cm
        cadd[...] = ca

    @pl.when(j == 1)
    def _():
        g = jnp.maximum(z * cmul[...] + cadd[...], 0.0)
        h2 = jnp.dot(g, w2_ref[...], preferred_element_type=jnp.float32)
        hs = h2 * dinv_ref[...]
        out_ref[0, :, :] = hs[:, :HH]
        out_ref[1, :, :] = hs[:, HH:]


def _tc_mid(agg1, dinv, b1, gn1_w, gn1_b, gn1_a, W2):
    return pl.pallas_call(
        _mid_body,
        grid=(2, NBLK),
        in_specs=[
            pl.BlockSpec((2, RB, HH), lambda j, i: (0, i, 0)),
            pl.BlockSpec((RB, 1), lambda j, i: (i, 0)),
            pl.BlockSpec((1, H), lambda j, i: (0, 0)),
            pl.BlockSpec((1, H), lambda j, i: (0, 0)),
            pl.BlockSpec((1, H), lambda j, i: (0, 0)),
            pl.BlockSpec((1, H), lambda j, i: (0, 0)),
            pl.BlockSpec((H, H), lambda j, i: (0, 0)),
        ],
        out_specs=pl.BlockSpec((2, RB, HH), lambda j, i: (0, i, 0)),
        out_shape=jax.ShapeDtypeStruct((2, NP, HH), jnp.float32),
        scratch_shapes=[
            pltpu.VMEM((1, H), jnp.float32),
            pltpu.VMEM((1, H), jnp.float32),
            pltpu.VMEM((1, H), jnp.float32),
            pltpu.VMEM((1, H), jnp.float32),
        ],
        compiler_params=pltpu.CompilerParams(
            dimension_semantics=("arbitrary", "arbitrary")),
    )(agg1, dinv, b1.reshape(1, H), gn1_w.reshape(1, H),
      gn1_b.reshape(1, H), gn1_a.reshape(1, H), W2)


def _fin_body(agg_ref, dinv_ref, b2_ref, w_ref, bb_ref, a_ref, x_ref,
              out_ref, s1, s2, cmul, cadd):
    j = pl.program_id(0)
    i = pl.program_id(1)

    @pl.when(jnp.logical_and(j == 0, i == 0))
    def _():
        s1[...] = jnp.zeros_like(s1)
        s2[...] = jnp.zeros_like(s2)

    z = _z_block(agg_ref, dinv_ref, b2_ref)

    @pl.when(j == 0)
    def _():
        s1[...] += jnp.sum(z, axis=0, keepdims=True)
        s2[...] += jnp.sum(z * z, axis=0, keepdims=True)

    @pl.when(jnp.logical_and(j == 1, i == 0))
    def _():
        cm, ca = _gn_coeffs(s1, s2, w_ref, bb_ref, a_ref)
        cmul[...] = cm
        cadd[...] = ca

    @pl.when(j == 1)
    def _():
        out_ref[...] = jnp.maximum(z * cmul[...] + cadd[...] + x_ref[...], 0.0)


def _tc_fin(agg2, dinv, b2, gn2_w, gn2_b, gn2_a, x):
    return pl.pallas_call(
        _fin_body,
        grid=(2, NBLK),
        in_specs=[
            pl.BlockSpec((2, RB, HH), lambda j, i: (0, i, 0)),
            pl.BlockSpec((RB, 1), lambda j, i: (i, 0)),
            pl.BlockSpec((1, H), lambda j, i: (0, 0)),
            pl.BlockSpec((1, H), lambda j, i: (0, 0)),
            pl.BlockSpec((1, H), lambda j, i: (0, 0)),
            pl.BlockSpec((1, H), lambda j, i: (0, 0)),
            pl.BlockSpec((RB, H), lambda j, i: (i, 0)),
        ],
        out_specs=pl.BlockSpec((RB, H), lambda j, i: (i, 0)),
        out_shape=jax.ShapeDtypeStruct((N, H), jnp.float32),
        scratch_shapes=[
            pltpu.VMEM((1, H), jnp.float32),
            pltpu.VMEM((1, H), jnp.float32),
            pltpu.VMEM((1, H), jnp.float32),
            pltpu.VMEM((1, H), jnp.float32),
        ],
        compiler_params=pltpu.CompilerParams(
            dimension_semantics=("arbitrary", "arbitrary")),
    )(agg2, dinv, b2.reshape(1, H), gn2_w.reshape(1, H),
      gn2_b.reshape(1, H), gn2_a.reshape(1, H), x)


def _conv_jnp(hs, src, dst):
    return hs.at[:, dst, :].add(hs[:, src, :])


def kernel(x, edge_index, W1, b1, W2, b2, gn1_w, gn1_b, gn1_a, gn2_w, gn2_b, gn2_a):
    pad = jnp.full((2, EP - E), N, dtype=edge_index.dtype)
    ei = jnp.concatenate([edge_index, pad], axis=1)
    src = ei[0].reshape(NS, CH, CW)
    dst = ei[1].reshape(NS, CH, CW)
    zeros128 = jnp.zeros((NP, 128), jnp.float32)
    ones128 = jnp.ones((CW, 128), jnp.float32)

    degp = _deg_kernel(dst, zeros128, ones128)
    hs1, dinv = _tc_pre(x, W1, degp)
    agg1 = _conv_kernel(hs1, src, dst)
    hs2 = _tc_mid(agg1, dinv, b1, gn1_w, gn1_b, gn1_a, W2)
    agg2 = _conv_kernel(hs2, src, dst)
    return _tc_fin(agg2, dinv, b2, gn2_w, gn2_b, gn2_a, x)


# speedup vs baseline: 1.4751x; 1.0272x over previous
import functools

import jax
import jax.numpy as jnp
from jax import lax
from jax.experimental import pallas as pl
from jax.experimental.pallas import tpu as pltpu
from jax.experimental.pallas import tpu_sc as plsc

N = 10000
NP = 10240
E = 160000
EP = 163840
D = 256
H = 256
HH = H // 2
NC, NS = 2, 16
CW = 128
EPT = EP // NS
CH = EPT // CW
CHD = EPT // NC // CW
RPT = NP // NS
RB = 1000
NBLK = N // RB


def _sc_mesh():
    return plsc.VectorSubcoreMesh(core_axis_name="c", subcore_axis_name="s")


CHD = (EP // (NC * NS)) // CW


@functools.partial(
    pl.kernel,
    out_type=jax.ShapeDtypeStruct((NC, NP, 128), jnp.float32),
    mesh=_sc_mesh(),
    scratch_types=(
        [pltpu.VMEM((CHD, CW), jnp.int32)]
        + [pltpu.VMEM((CW,), jnp.int32)] * 4
        + [pltpu.VMEM((CW, 128), jnp.float32)]
        + [pltpu.VMEM_SHARED((NP, 128), jnp.float32)]
        + [pltpu.SemaphoreType.DMA] * 4
    ),
)
def _deg_kernel(dst_hbm, zeros_hbm, ones_hbm, out_hbm, dst_v, *rest):
    dcur = rest[0:4]
    ones_v = rest[4]
    acc = rest[5]
    sems = rest[6:10]
    c = lax.axis_index("c")
    s = lax.axis_index("s")
    pltpu.sync_copy(dst_hbm.at[s, pl.ds(c * CHD, CHD)], dst_v)
    pltpu.sync_copy(ones_hbm, ones_v)
    pltpu.sync_copy(zeros_hbm.at[pl.ds(s * RPT, RPT)], acc.at[pl.ds(s * RPT, RPT)])
    plsc.subcore_barrier()

    def body(g, carry):
        sds = []
        for b in range(4):
            k = g * 4 + b
            for i in range(CW // 16):
                dcur[b][pl.ds(i * 16, 16)] = dst_v[k, pl.ds(i * 16, 16)]
            sds.append(pltpu.async_copy(ones_v, acc.at[dcur[b]], sems[b], add=True))
        for b in range(4):
            sds[b].wait()
        return carry

    lax.fori_loop(0, CHD // 4, body, 0)
    plsc.subcore_barrier()
    pltpu.sync_copy(acc.at[pl.ds(s * RPT, RPT)], out_hbm.at[c, pl.ds(s * RPT, RPT)])


NB = 4
SUB = 64
LAG = 3
GRP = 16
SG = GRP * CW // SUB


@functools.partial(
    pl.kernel,
    out_type=jax.ShapeDtypeStruct((NC, NP, HH), jnp.float32),
    mesh=_sc_mesh(),
    scratch_types=(
        [pltpu.VMEM((GRP, CW), jnp.int32)] * 2
        + [pltpu.VMEM((SUB,), jnp.int32)] * (2 * NB)
        + [pltpu.VMEM((SUB, HH), jnp.float32)] * NB
        + [pltpu.VMEM_SHARED((NP, HH), jnp.float32)]
        + [pltpu.SemaphoreType.DMA] * (2 * NB)
    ),
)
def _conv_kernel(hs_hbm, src_hbm, dst_hbm, out_hbm, sring, dring, *rest):
    scur = rest[0:NB]
    dcur = rest[NB:2 * NB]
    gbuf = rest[2 * NB:3 * NB]
    acc = rest[3 * NB]
    gsem = rest[3 * NB + 1:3 * NB + 1 + NB]
    ssem = rest[3 * NB + 1 + NB:3 * NB + 1 + 2 * NB]
    c = lax.axis_index("c")
    s = lax.axis_index("s")
    pltpu.sync_copy(hs_hbm.at[c, pl.ds(s * RPT, RPT)], acc.at[pl.ds(s * RPT, RPT)])
    plsc.subcore_barrier()
    table = hs_hbm.at[c]

    def fire_scatter(gds, sds, q):
        b2 = q % NB
        gds[b2].wait()
        sds[b2] = pltpu.async_copy(gbuf[b2], acc.at[dcur[b2]], ssem[b2], add=True)

    def group(gg, carry):
        pltpu.sync_copy(src_hbm.at[s, pl.ds(gg * GRP, GRP)], sring)
        pltpu.sync_copy(dst_hbm.at[s, pl.ds(gg * GRP, GRP)], dring)
        gds = [None] * NB
        sds = [None] * NB
        for q in range(SG):
            b = q % NB
            if sds[b] is not None:
                sds[b].wait()
            kk, off = q // (CW // SUB), (q % (CW // SUB)) * SUB
            for i in range(SUB // 16):
                scur[b][pl.ds(i * 16, 16)] = sring[kk, pl.ds(off + i * 16, 16)]
                dcur[b][pl.ds(i * 16, 16)] = dring[kk, pl.ds(off + i * 16, 16)]
            gds[b] = pltpu.async_copy(table.at[scur[b]], gbuf[b], gsem[b])
            if q >= LAG:
                fire_scatter(gds, sds, q - LAG)
        for q in range(SG - LAG, SG):
            fire_scatter(gds, sds, q)
        for b in range(NB):
            sds[b].wait()
        return carry

    lax.fori_loop(0, CH // GRP, group, 0)
    plsc.subcore_barrier()
    pltpu.sync_copy(acc.at[pl.ds(s * RPT, RPT)], out_hbm.at[c, pl.ds(s * RPT, RPT)])


def _dinv_block(degp):
    deg = degp[0, :, 0:1] + degp[1, :, 0:1] + 1.0
    return lax.rsqrt(deg)


def _pre_body(x_ref, w_ref, degp_ref, out_ref, dinv_ref):
    h = jnp.dot(x_ref[...], w_ref[...], preferred_element_type=jnp.float32)
    dinv = _dinv_block(degp_ref[...])
    hs = h * dinv
    out_ref[0, :, :] = hs[:, :HH]
    out_ref[1, :, :] = hs[:, HH:]
    dinv_ref[...] = dinv


def _tc_pre(x, W1, degp):
    return pl.pallas_call(
        _pre_body,
        grid=(NBLK,),
        in_specs=[
            pl.BlockSpec((RB, D), lambda i: (i, 0)),
            pl.BlockSpec((D, H), lambda i: (0, 0)),
            pl.BlockSpec((2, RB, HH), lambda i: (0, i, 0)),
        ],
        out_specs=[
            pl.BlockSpec((2, RB, HH), lambda i: (0, i, 0)),
            pl.BlockSpec((RB, 1), lambda i: (i, 0)),
        ],
        out_shape=[
            jax.ShapeDtypeStruct((2, NP, HH), jnp.float32),
            jax.ShapeDtypeStruct((N, 1), jnp.float32),
        ],
    )(x, W1, degp)


def _z_block(agg_ref, dinv_ref, b_ref):
    agg = jnp.concatenate([agg_ref[0], agg_ref[1]], axis=1)
    return agg * dinv_ref[...] + b_ref[...]


def _gn_coeffs(s1_ref, s2_ref, w_ref, b_ref, a_ref, eps=1e-5):
    m = s1_ref[...] / N
    var = s2_ref[...] / N - m * m * a_ref[...] * (2.0 - a_ref[...])
    cmul = w_ref[...] * lax.rsqrt(var + eps)
    cadd = b_ref[...] - a_ref[...] * m * cmul
    return cmul, cadd


def _mid_body(agg_ref, dinv_ref, b1_ref, w_ref, bb_ref, a_ref, w2_ref,
              out_ref, s1, s2, cmul, cadd):
    j = pl.program_id(0)
    i = pl.program_id(1)

    @pl.when(jnp.logical_and(j == 0, i == 0))
    def _():
        s1[...] = jnp.zeros_like(s1)
        s2[...] = jnp.zeros_like(s2)

    z = _z_block(agg_ref, dinv_ref, b1_ref)

    @pl.when(j == 0)
    def _():
        s1[...] += jnp.sum(z, axis=0, keepdims=True)
        s2[...] += jnp.sum(z * z, axis=0, keepdims=True)

    @pl.when(jnp.logical_and(j == 1, i == 0))
    def _():
        cm, ca = _gn_coeffs(s1, s2, w_ref, bb_ref, a_ref)
        cmul[...] = cm
        cadd[...] = ca

    @pl.when(j == 1)
    def _():
        g = jnp.maximum(z * cmul[...] + cadd[...], 0.0)
        h2 = jnp.dot(g, w2_ref[...], preferred_element_type=jnp.float32)
        hs = h2 * dinv_ref[...]
        out_ref[0, :, :] = hs[:, :HH]
        out_ref[1, :, :] = hs[:, HH:]


def _tc_mid(agg1, dinv, b1, gn1_w, gn1_b, gn1_a, W2):
    return pl.pallas_call(
        _mid_body,
        grid=(2, NBLK),
        in_specs=[
            pl.BlockSpec((2, RB, HH), lambda j, i: (0, i, 0)),
            pl.BlockSpec((RB, 1), lambda j, i: (i, 0)),
            pl.BlockSpec((1, H), lambda j, i: (0, 0)),
            pl.BlockSpec((1, H), lambda j, i: (0, 0)),
            pl.BlockSpec((1, H), lambda j, i: (0, 0)),
            pl.BlockSpec((1, H), lambda j, i: (0, 0)),
            pl.BlockSpec((H, H), lambda j, i: (0, 0)),
        ],
        out_specs=pl.BlockSpec((2, RB, HH), lambda j, i: (0, i, 0)),
        out_shape=jax.ShapeDtypeStruct((2, NP, HH), jnp.float32),
        scratch_shapes=[
            pltpu.VMEM((1, H), jnp.float32),
            pltpu.VMEM((1, H), jnp.float32),
            pltpu.VMEM((1, H), jnp.float32),
            pltpu.VMEM((1, H), jnp.float32),
        ],
        compiler_params=pltpu.CompilerParams(
            dimension_semantics=("arbitrary", "arbitrary")),
    )(agg1, dinv, b1.reshape(1, H), gn1_w.reshape(1, H),
      gn1_b.reshape(1, H), gn1_a.reshape(1, H), W2)


def _fin_body(agg_ref, dinv_ref, b2_ref, w_ref, bb_ref, a_ref, x_ref,
              out_ref, s1, s2, cmul, cadd):
    j = pl.program_id(0)
    i = pl.program_id(1)

    @pl.when(jnp.logical_and(j == 0, i == 0))
    def _():
        s1[...] = jnp.zeros_like(s1)
        s2[...] = jnp.zeros_like(s2)

    z = _z_block(agg_ref, dinv_ref, b2_ref)

    @pl.when(j == 0)
    def _():
        s1[...] += jnp.sum(z, axis=0, keepdims=True)
        s2[...] += jnp.sum(z * z, axis=0, keepdims=True)

    @pl.when(jnp.logical_and(j == 1, i == 0))
    def _():
        cm, ca = _gn_coeffs(s1, s2, w_ref, bb_ref, a_ref)
        cmul[...] = cm
        cadd[...] = ca

    @pl.when(j == 1)
    def _():
        out_ref[...] = jnp.maximum(z * cmul[...] + cadd[...] + x_ref[...], 0.0)


def _tc_fin(agg2, dinv, b2, gn2_w, gn2_b, gn2_a, x):
    return pl.pallas_call(
        _fin_body,
        grid=(2, NBLK),
        in_specs=[
            pl.BlockSpec((2, RB, HH), lambda j, i: (0, i, 0)),
            pl.BlockSpec((RB, 1), lambda j, i: (i, 0)),
            pl.BlockSpec((1, H), lambda j, i: (0, 0)),
            pl.BlockSpec((1, H), lambda j, i: (0, 0)),
            pl.BlockSpec((1, H), lambda j, i: (0, 0)),
            pl.BlockSpec((1, H), lambda j, i: (0, 0)),
            pl.BlockSpec((RB, H), lambda j, i: (i, 0)),
        ],
        out_specs=pl.BlockSpec((RB, H), lambda j, i: (i, 0)),
        out_shape=jax.ShapeDtypeStruct((N, H), jnp.float32),
        scratch_shapes=[
            pltpu.VMEM((1, H), jnp.float32),
            pltpu.VMEM((1, H), jnp.float32),
            pltpu.VMEM((1, H), jnp.float32),
            pltpu.VMEM((1, H), jnp.float32),
        ],
        compiler_params=pltpu.CompilerParams(
            dimension_semantics=("arbitrary", "arbitrary")),
    )(agg2, dinv, b2.reshape(1, H), gn2_w.reshape(1, H),
      gn2_b.reshape(1, H), gn2_a.reshape(1, H), x)


def _conv_jnp(hs, src, dst):
    return hs.at[:, dst, :].add(hs[:, src, :])


def kernel(x, edge_index, W1, b1, W2, b2, gn1_w, gn1_b, gn1_a, gn2_w, gn2_b, gn2_a):
    pad = jnp.full((2, EP - E), N, dtype=edge_index.dtype)
    ei = jnp.concatenate([edge_index, pad], axis=1)
    src = ei[0].reshape(NS, CH, CW)
    dst = ei[1].reshape(NS, CH, CW)
    zeros128 = jnp.zeros((NP, 128), jnp.float32)
    ones128 = jnp.ones((CW, 128), jnp.float32)

    degp = _deg_kernel(dst, zeros128, ones128)
    hs1, dinv = _tc_pre(x, W1, degp)
    agg1 = _conv_kernel(hs1, src, dst)
    hs2 = _tc_mid(agg1, dinv, b1, gn1_w, gn1_b, gn1_a, W2)
    agg2 = _conv_kernel(hs2, src, dst)
    return _tc_fin(agg2, dinv, b2, gn2_w, gn2_b, gn2_a, x)


# speedup vs baseline: 1.4941x; 1.0129x over previous
import functools

import jax
import jax.numpy as jnp
from jax import lax
from jax.experimental import pallas as pl
from jax.experimental.pallas import tpu as pltpu
from jax.experimental.pallas import tpu_sc as plsc

N = 10000
NP = 10240
E = 160000
EP = 163840
D = 256
H = 256
HH = H // 2
NC, NS = 2, 16
CW = 128
EPT = EP // NS
CH = EPT // CW
CHD = EPT // NC // CW
RPT = NP // NS
RB = 1000
NBLK = N // RB


def _sc_mesh():
    return plsc.VectorSubcoreMesh(core_axis_name="c", subcore_axis_name="s")


CHD = (EP // (NC * NS)) // CW


@functools.partial(
    pl.kernel,
    out_type=jax.ShapeDtypeStruct((NC, NP, 128), jnp.float32),
    mesh=_sc_mesh(),
    scratch_types=(
        [pltpu.VMEM((CHD, CW), jnp.int32)]
        + [pltpu.VMEM((CW,), jnp.int32)] * 4
        + [pltpu.VMEM((CW, 128), jnp.float32)]
        + [pltpu.VMEM_SHARED((NP, 128), jnp.float32)]
        + [pltpu.SemaphoreType.DMA] * 4
    ),
)
def _deg_kernel(dst_hbm, zeros_hbm, ones_hbm, out_hbm, dst_v, *rest):
    dcur = rest[0:4]
    ones_v = rest[4]
    acc = rest[5]
    sems = rest[6:10]
    c = lax.axis_index("c")
    s = lax.axis_index("s")
    pltpu.sync_copy(dst_hbm.at[s, pl.ds(c * CHD, CHD)], dst_v)
    pltpu.sync_copy(ones_hbm, ones_v)
    pltpu.sync_copy(zeros_hbm.at[pl.ds(s * RPT, RPT)], acc.at[pl.ds(s * RPT, RPT)])
    plsc.subcore_barrier()

    def body(g, carry):
        sds = []
        for b in range(4):
            k = g * 4 + b
            for i in range(CW // 16):
                dcur[b][pl.ds(i * 16, 16)] = dst_v[k, pl.ds(i * 16, 16)]
            sds.append(pltpu.async_copy(ones_v, acc.at[dcur[b]], sems[b], add=True))
        for b in range(4):
            sds[b].wait()
        return carry

    lax.fori_loop(0, CHD // 4, body, 0)
    plsc.subcore_barrier()
    pltpu.sync_copy(acc.at[pl.ds(s * RPT, RPT)], out_hbm.at[c, pl.ds(s * RPT, RPT)])


NB = 4
SUB = 64
LAG = 3
GRP = 40
SG = GRP * CW // SUB


@functools.partial(
    pl.kernel,
    out_type=jax.ShapeDtypeStruct((NC, NP, HH), jnp.float32),
    mesh=_sc_mesh(),
    scratch_types=(
        [pltpu.VMEM((GRP, CW), jnp.int32)] * 2
        + [pltpu.VMEM((SUB,), jnp.int32)] * (2 * NB)
        + [pltpu.VMEM((SUB, HH), jnp.float32)] * NB
        + [pltpu.VMEM_SHARED((NP, HH), jnp.float32)]
        + [pltpu.SemaphoreType.DMA] * (2 * NB)
    ),
)
def _conv_kernel(hs_hbm, src_hbm, dst_hbm, out_hbm, sring, dring, *rest):
    scur = rest[0:NB]
    dcur = rest[NB:2 * NB]
    gbuf = rest[2 * NB:3 * NB]
    acc = rest[3 * NB]
    gsem = rest[3 * NB + 1:3 * NB + 1 + NB]
    ssem = rest[3 * NB + 1 + NB:3 * NB + 1 + 2 * NB]
    c = lax.axis_index("c")
    s = lax.axis_index("s")
    pltpu.sync_copy(hs_hbm.at[c, pl.ds(s * RPT, RPT)], acc.at[pl.ds(s * RPT, RPT)])
    plsc.subcore_barrier()
    table = hs_hbm.at[c]

    def fire_scatter(gds, sds, q):
        b2 = q % NB
        gds[b2].wait()
        sds[b2] = pltpu.async_copy(gbuf[b2], acc.at[dcur[b2]], ssem[b2], add=True)

    def group(gg, carry):
        pltpu.sync_copy(src_hbm.at[s, pl.ds(gg * GRP, GRP)], sring)
        pltpu.sync_copy(dst_hbm.at[s, pl.ds(gg * GRP, GRP)], dring)
        gds = [None] * NB
        sds = [None] * NB
        for q in range(SG):
            b = q % NB
            if sds[b] is not None:
                sds[b].wait()
            kk, off = q // (CW // SUB), (q % (CW // SUB)) * SUB
            for i in range(SUB // 16):
                scur[b][pl.ds(i * 16, 16)] = sring[kk, pl.ds(off + i * 16, 16)]
                dcur[b][pl.ds(i * 16, 16)] = dring[kk, pl.ds(off + i * 16, 16)]
            gds[b] = pltpu.async_copy(table.at[scur[b]], gbuf[b], gsem[b])
            if q >= LAG:
                fire_scatter(gds, sds, q - LAG)
        for q in range(SG - LAG, SG):
            fire_scatter(gds, sds, q)
        for b in range(NB):
            sds[b].wait()
        return carry

    lax.fori_loop(0, CH // GRP, group, 0)
    plsc.subcore_barrier()
    pltpu.sync_copy(acc.at[pl.ds(s * RPT, RPT)], out_hbm.at[c, pl.ds(s * RPT, RPT)])


def _dinv_block(degp):
    deg = degp[0, :, 0:1] + degp[1, :, 0:1] + 1.0
    return lax.rsqrt(deg)


def _pre_body(x_ref, w_ref, degp_ref, out_ref, dinv_ref):
    h = jnp.dot(x_ref[...], w_ref[...], preferred_element_type=jnp.float32)
    dinv = _dinv_block(degp_ref[...])
    hs = h * dinv
    out_ref[0, :, :] = hs[:, :HH]
    out_ref[1, :, :] = hs[:, HH:]
    dinv_ref[...] = dinv


def _tc_pre(x, W1, degp):
    return pl.pallas_call(
        _pre_body,
        grid=(NBLK,),
        in_specs=[
            pl.BlockSpec((RB, D), lambda i: (i, 0)),
            pl.BlockSpec((D, H), lambda i: (0, 0)),
            pl.BlockSpec((2, RB, HH), lambda i: (0, i, 0)),
        ],
        out_specs=[
            pl.BlockSpec((2, RB, HH), lambda i: (0, i, 0)),
            pl.BlockSpec((RB, 1), lambda i: (i, 0)),
        ],
        out_shape=[
            jax.ShapeDtypeStruct((2, NP, HH), jnp.float32),
            jax.ShapeDtypeStruct((N, 1), jnp.float32),
        ],
    )(x, W1, degp)


def _z_block(agg_ref, dinv_ref, b_ref):
    agg = jnp.concatenate([agg_ref[0], agg_ref[1]], axis=1)
    return agg * dinv_ref[...] + b_ref[...]


def _gn_coeffs(s1_ref, s2_ref, w_ref, b_ref, a_ref, eps=1e-5):
    m = s1_ref[...] / N
    var = s2_ref[...] / N - m * m * a_ref[...] * (2.0 - a_ref[...])
    cmul = w_ref[...] * lax.rsqrt(var + eps)
    cadd = b_ref[...] - a_ref[...] * m * cmul
    return cmul, cadd


def _mid_body(agg_ref, dinv_ref, b1_ref, w_ref, bb_ref, a_ref, w2_ref,
              out_ref, s1, s2, cmul, cadd):
    j = pl.program_id(0)
    i = pl.program_id(1)

    @pl.when(jnp.logical_and(j == 0, i == 0))
    def _():
        s1[...] = jnp.zeros_like(s1)
        s2[...] = jnp.zeros_like(s2)

    z = _z_block(agg_ref, dinv_ref, b1_ref)

    @pl.when(j == 0)
    def _():
        s1[...] += jnp.sum(z, axis=0, keepdims=True)
        s2[...] += jnp.sum(z * z, axis=0, keepdims=True)

    @pl.when(jnp.logical_and(j == 1, i == 0))
    def _():
        cm, ca = _gn_coeffs(s1, s2, w_ref, bb_ref, a_ref)
        cmul[...] = cm
        cadd[...] = ca

    @pl.when(j == 1)
    def _():
        g = jnp.maximum(z * cmul[...] + cadd[...], 0.0)
        h2 = jnp.dot(g, w2_ref[...], preferred_element_type=jnp.float32)
        hs = h2 * dinv_ref[...]
        out_ref[0, :, :] = hs[:, :HH]
        out_ref[1, :, :] = hs[:, HH:]


def _tc_mid(agg1, dinv, b1, gn1_w, gn1_b, gn1_a, W2):
    return pl.pallas_call(
        _mid_body,
        grid=(2, NBLK),
        in_specs=[
            pl.BlockSpec((2, RB, HH), lambda j, i: (0, i, 0)),
            pl.BlockSpec((RB, 1), lambda j, i: (i, 0)),
            pl.BlockSpec((1, H), lambda j, i: (0, 0)),
            pl.BlockSpec((1, H), lambda j, i: (0, 0)),
            pl.BlockSpec((1, H), lambda j, i: (0, 0)),
            pl.BlockSpec((1, H), lambda j, i: (0, 0)),
            pl.BlockSpec((H, H), lambda j, i: (0, 0)),
        ],
        out_specs=pl.BlockSpec((2, RB, HH), lambda j, i: (0, i, 0)),
        out_shape=jax.ShapeDtypeStruct((2, NP, HH), jnp.float32),
        scratch_shapes=[
            pltpu.VMEM((1, H), jnp.float32),
            pltpu.VMEM((1, H), jnp.float32),
            pltpu.VMEM((1, H), jnp.float32),
            pltpu.VMEM((1, H), jnp.float32),
        ],
        compiler_params=pltpu.CompilerParams(
            dimension_semantics=("arbitrary", "arbitrary")),
    )(agg1, dinv, b1.reshape(1, H), gn1_w.reshape(1, H),
      gn1_b.reshape(1, H), gn1_a.reshape(1, H), W2)


def _fin_body(agg_ref, dinv_ref, b2_ref, w_ref, bb_ref, a_ref, x_ref,
              out_ref, s1, s2, cmul, cadd):
    j = pl.program_id(0)
    i = pl.program_id(1)

    @pl.when(jnp.logical_and(j == 0, i == 0))
    def _():
        s1[...] = jnp.zeros_like(s1)
        s2[...] = jnp.zeros_like(s2)

    z = _z_block(agg_ref, dinv_ref, b2_ref)

    @pl.when(j == 0)
    def _():
        s1[...] += jnp.sum(z, axis=0, keepdims=True)
        s2[...] += jnp.sum(z * z, axis=0, keepdims=True)

    @pl.when(jnp.logical_and(j == 1, i == 0))
    def _():
        cm, ca = _gn_coeffs(s1, s2, w_ref, bb_ref, a_ref)
        cmul[...] = cm
        cadd[...] = ca

    @pl.when(j == 1)
    def _():
        out_ref[...] = jnp.maximum(z * cmul[...] + cadd[...] + x_ref[...], 0.0)


def _tc_fin(agg2, dinv, b2, gn2_w, gn2_b, gn2_a, x):
    return pl.pallas_call(
        _fin_body,
        grid=(2, NBLK),
        in_specs=[
            pl.BlockSpec((2, RB, HH), lambda j, i: (0, i, 0)),
            pl.BlockSpec((RB, 1), lambda j, i: (i, 0)),
            pl.BlockSpec((1, H), lambda j, i: (0, 0)),
            pl.BlockSpec((1, H), lambda j, i: (0, 0)),
            pl.BlockSpec((1, H), lambda j, i: (0, 0)),
            pl.BlockSpec((1, H), lambda j, i: (0, 0)),
            pl.BlockSpec((RB, H), lambda j, i: (i, 0)),
        ],
        out_specs=pl.BlockSpec((RB, H), lambda j, i: (i, 0)),
        out_shape=jax.ShapeDtypeStruct((N, H), jnp.float32),
        scratch_shapes=[
            pltpu.VMEM((1, H), jnp.float32),
            pltpu.VMEM((1, H), jnp.float32),
            pltpu.VMEM((1, H), jnp.float32),
            pltpu.VMEM((1, H), jnp.float32),
        ],
        compiler_params=pltpu.CompilerParams(
            dimension_semantics=("arbitrary", "arbitrary")),
    )(agg2, dinv, b2.reshape(1, H), gn2_w.reshape(1, H),
      gn2_b.reshape(1, H), gn2_a.reshape(1, H), x)


def _conv_jnp(hs, src, dst):
    return hs.at[:, dst, :].add(hs[:, src, :])


def kernel(x, edge_index, W1, b1, W2, b2, gn1_w, gn1_b, gn1_a, gn2_w, gn2_b, gn2_a):
    pad = jnp.full((2, EP - E), N, dtype=edge_index.dtype)
    ei = jnp.concatenate([edge_index, pad], axis=1)
    src = ei[0].reshape(NS, CH, CW)
    dst = ei[1].reshape(NS, CH, CW)
    zeros128 = jnp.zeros((NP, 128), jnp.float32)
    ones128 = jnp.ones((CW, 128), jnp.float32)

    degp = _deg_kernel(dst, zeros128, ones128)
    hs1, dinv = _tc_pre(x, W1, degp)
    agg1 = _conv_kernel(hs1, src, dst)
    hs2 = _tc_mid(agg1, dinv, b1, gn1_w, gn1_b, gn1_a, W2)
    agg2 = _conv_kernel(hs2, src, dst)
    return _tc_fin(agg2, dinv, b2, gn2_w, gn2_b, gn2_a, x)


# speedup vs baseline: 1.4942x; 1.0001x over previous
import functools

import jax
import jax.numpy as jnp
from jax import lax
from jax.experimental import pallas as pl
from jax.experimental.pallas import tpu as pltpu
from jax.experimental.pallas import tpu_sc as plsc

N = 10000
NP = 10240
E = 160000
EP = 163840
D = 256
H = 256
HH = H // 2
NC, NS = 2, 16
CW = 128
EPT = EP // NS
CH = EPT // CW
CHD = EPT // NC // CW
RPT = NP // NS
RB = 1000
NBLK = N // RB


def _sc_mesh():
    return plsc.VectorSubcoreMesh(core_axis_name="c", subcore_axis_name="s")


CHD = (EP // (NC * NS)) // CW


@functools.partial(
    pl.kernel,
    out_type=jax.ShapeDtypeStruct((NC, NP, 128), jnp.float32),
    mesh=_sc_mesh(),
    scratch_types=(
        [pltpu.VMEM((CHD, CW), jnp.int32)]
        + [pltpu.VMEM((CW,), jnp.int32)] * 4
        + [pltpu.VMEM((CW, 128), jnp.float32)]
        + [pltpu.VMEM_SHARED((NP, 128), jnp.float32)]
        + [pltpu.SemaphoreType.DMA] * 4
    ),
)
def _deg_kernel(dst_hbm, zeros_hbm, ones_hbm, out_hbm, dst_v, *rest):
    dcur = rest[0:4]
    ones_v = rest[4]
    acc = rest[5]
    sems = rest[6:10]
    c = lax.axis_index("c")
    s = lax.axis_index("s")
    pltpu.sync_copy(dst_hbm.at[s, pl.ds(c * CHD, CHD)], dst_v)
    pltpu.sync_copy(ones_hbm, ones_v)
    pltpu.sync_copy(zeros_hbm.at[pl.ds(s * RPT, RPT)], acc.at[pl.ds(s * RPT, RPT)])
    plsc.subcore_barrier()

    def body(g, carry):
        sds = []
        for b in range(4):
            k = g * 4 + b
            for i in range(CW // 16):
                dcur[b][pl.ds(i * 16, 16)] = dst_v[k, pl.ds(i * 16, 16)]
            sds.append(pltpu.async_copy(ones_v, acc.at[dcur[b]], sems[b], add=True))
        for b in range(4):
            sds[b].wait()
        return carry

    lax.fori_loop(0, CHD // 4, body, 0)
    plsc.subcore_barrier()
    pltpu.sync_copy(acc.at[pl.ds(s * RPT, RPT)], out_hbm.at[c, pl.ds(s * RPT, RPT)])


NB = 4
SUB = 64
LAG = 3
GRP = 40
SG = GRP * CW // SUB


@functools.partial(
    pl.kernel,
    out_type=jax.ShapeDtypeStruct((NC, NP, HH), jnp.float32),
    mesh=_sc_mesh(),
    scratch_types=(
        [pltpu.VMEM((GRP, CW), jnp.int32)] * 2
        + [pltpu.VMEM((SUB,), jnp.int32)] * (2 * NB)
        + [pltpu.VMEM((SUB, HH), jnp.float32)] * NB
        + [pltpu.VMEM_SHARED((NP, HH), jnp.float32)]
        + [pltpu.SemaphoreType.DMA] * (2 * NB)
    ),
)
def _conv_kernel(hs_hbm, src_hbm, dst_hbm, out_hbm, sring, dring, *rest):
    scur = rest[0:NB]
    dcur = rest[NB:2 * NB]
    gbuf = rest[2 * NB:3 * NB]
    acc = rest[3 * NB]
    gsem = rest[3 * NB + 1:3 * NB + 1 + NB]
    ssem = rest[3 * NB + 1 + NB:3 * NB + 1 + 2 * NB]
    c = lax.axis_index("c")
    s = lax.axis_index("s")
    pltpu.sync_copy(hs_hbm.at[c, pl.ds(s * RPT, RPT)], acc.at[pl.ds(s * RPT, RPT)])
    plsc.subcore_barrier()
    table = hs_hbm.at[c]

    def fire_scatter(gds, sds, q):
        b2 = q % NB
        gds[b2].wait()
        sds[b2] = pltpu.async_copy(gbuf[b2], acc.at[dcur[b2]], ssem[b2], add=True)

    def group(gg, carry):
        pltpu.sync_copy(src_hbm.at[s, pl.ds(gg * GRP, GRP)], sring)
        pltpu.sync_copy(dst_hbm.at[s, pl.ds(gg * GRP, GRP)], dring)
        gds = [None] * NB
        sds = [None] * NB
        for q in range(SG):
            b = q % NB
            if sds[b] is not None:
                sds[b].wait()
            kk, off = q // (CW // SUB), (q % (CW // SUB)) * SUB
            for i in range(SUB // 16):
                scur[b][pl.ds(i * 16, 16)] = sring[kk, pl.ds(off + i * 16, 16)]
                dcur[b][pl.ds(i * 16, 16)] = dring[kk, pl.ds(off + i * 16, 16)]
            gds[b] = pltpu.async_copy(table.at[scur[b]], gbuf[b], gsem[b])
            if q >= LAG:
                fire_scatter(gds, sds, q - LAG)
        for q in range(SG - LAG, SG):
            fire_scatter(gds, sds, q)
        for b in range(NB):
            sds[b].wait()
        return carry

    lax.fori_loop(0, CH // GRP, group, 0)
    plsc.subcore_barrier()
    pltpu.sync_copy(acc.at[pl.ds(s * RPT, RPT)], out_hbm.at[c, pl.ds(s * RPT, RPT)])


def _dinv_block(degp):
    deg = degp[0, :, 0:1] + degp[1, :, 0:1] + 1.0
    return lax.rsqrt(deg)


def _pre_body(x_ref, w_ref, degp_ref, out_ref, dinv_ref):
    h = jnp.dot(x_ref[...], w_ref[...], preferred_element_type=jnp.float32)
    dinv = _dinv_block(degp_ref[...])
    hs = h * dinv
    out_ref[0, :, :] = hs[:, :HH]
    out_ref[1, :, :] = hs[:, HH:]
    dinv_ref[...] = dinv


def _tc_pre(x, W1, degp):
    return pl.pallas_call(
        _pre_body,
        grid=(NBLK,),
        in_specs=[
            pl.BlockSpec((RB, D), lambda i: (i, 0)),
            pl.BlockSpec((D, H), lambda i: (0, 0)),
            pl.BlockSpec((2, RB, HH), lambda i: (0, i, 0)),
        ],
        out_specs=[
            pl.BlockSpec((2, RB, HH), lambda i: (0, i, 0)),
            pl.BlockSpec((RB, 1), lambda i: (i, 0)),
        ],
        out_shape=[
            jax.ShapeDtypeStruct((2, NP, HH), jnp.float32),
            jax.ShapeDtypeStruct((N, 1), jnp.float32),
        ],
    )(x, W1, degp)


def _z_block(agg_ref, dinv_ref, b_ref):
    agg = jnp.concatenate([agg_ref[0], agg_ref[1]], axis=1)
    return agg * dinv_ref[...] + b_ref[...]


def _gn_coeffs(s1_ref, s2_ref, w_ref, b_ref, a_ref, eps=1e-5):
    m = s1_ref[...] / N
    var = s2_ref[...] / N - m * m * a_ref[...] * (2.0 - a_ref[...])
    cmul = w_ref[...] * lax.rsqrt(var + eps)
    cadd = b_ref[...] - a_ref[...] * m * cmul
    return cmul, cadd


def _mid_body(agg_ref, dinv_ref, b1_ref, w_ref, bb_ref, a_ref, w2_ref,
              out_ref, s1, s2, cmul, cadd):
    j = pl.program_id(0)
    i = pl.program_id(1)

    @pl.when(jnp.logical_and(j == 0, i == 0))
    def _():
        s1[...] = jnp.zeros_like(s1)
        s2[...] = jnp.zeros_like(s2)

    z = _z_block(agg_ref, dinv_ref, b1_ref)

    @pl.when(j == 0)
    def _():
        s1[...] += jnp.sum(z, axis=0, keepdims=True)
        s2[...] += jnp.sum(z * z, axis=0, keepdims=True)

    @pl.when(jnp.logical_and(j == 1, i == 0))
    def _():
        cm, ca = _gn_coeffs(s1, s2, w_ref, bb_ref, a_ref)
        cmul[...] = cm
        cadd[...] = ca

    @pl.when(j == 1)
    def _():
        g = jnp.maximum(z * cmul[...] + cadd[...], 0.0)
        h2 = jnp.dot(g, w2_ref[...], preferred_element_type=jnp.float32)
        hs = h2 * dinv_ref[...]
        out_ref[0, :, :] = hs[:, :HH]
        out_ref[1, :, :] = hs[:, HH:]


def _tc_mid(agg1, dinv, b1, gn1_w, gn1_b, gn1_a, W2):
    return pl.pallas_call(
        _mid_body,
        grid=(2, NBLK),
        in_specs=[
            pl.BlockSpec((2, RB, HH), lambda j, i: (0, i, 0)),
            pl.BlockSpec((RB, 1), lambda j, i: (i, 0)),
            pl.BlockSpec((1, H), lambda j, i: (0, 0)),
            pl.BlockSpec((1, H), lambda j, i: (0, 0)),
            pl.BlockSpec((1, H), lambda j, i: (0, 0)),
            pl.BlockSpec((1, H), lambda j, i: (0, 0)),
            pl.BlockSpec((H, H), lambda j, i: (0, 0)),
        ],
        out_specs=pl.BlockSpec((2, RB, HH), lambda j, i: (0, i, 0)),
        out_shape=jax.ShapeDtypeStruct((2, NP, HH), jnp.float32),
        scratch_shapes=[
            pltpu.VMEM((1, H), jnp.float32),
            pltpu.VMEM((1, H), jnp.float32),
            pltpu.VMEM((1, H), jnp.float32),
            pltpu.VMEM((1, H), jnp.float32),
        ],
        compiler_params=pltpu.CompilerParams(
            dimension_semantics=("arbitrary", "arbitrary")),
    )(agg1, dinv, b1.reshape(1, H), gn1_w.reshape(1, H),
      gn1_b.reshape(1, H), gn1_a.reshape(1, H), W2)


def _fin_body(agg_ref, dinv_ref, b2_ref, w_ref, bb_ref, a_ref, x_ref,
              out_ref, s1, s2, cmul, cadd):
    j = pl.program_id(0)
    i = pl.program_id(1)

    @pl.when(jnp.logical_and(j == 0, i == 0))
    def _():
        s1[...] = jnp.zeros_like(s1)
        s2[...] = jnp.zeros_like(s2)

    z = _z_block(agg_ref, dinv_ref, b2_ref)

    @pl.when(j == 0)
    def _():
        s1[...] += jnp.sum(z, axis=0, keepdims=True)
        s2[...] += jnp.sum(z * z, axis=0, keepdims=True)

    @pl.when(jnp.logical_and(j == 1, i == 0))
    def _():
        cm, ca = _gn_coeffs(s1, s2, w_ref, bb_ref, a_ref)
        cmul[...] = cm
        cadd[...] = ca

    @pl.when(j == 1)
    def _():
        out_ref[...] = jnp.maximum(z * cmul[...] + cadd[...] + x_ref[...], 0.0)


def _tc_fin(agg2, dinv, b2, gn2_w, gn2_b, gn2_a, x):
    return pl.pallas_call(
        _fin_body,
        grid=(2, NBLK),
        in_specs=[
            pl.BlockSpec((2, RB, HH), lambda j, i: (0, i, 0)),
            pl.BlockSpec((RB, 1), lambda j, i: (i, 0)),
            pl.BlockSpec((1, H), lambda j, i: (0, 0)),
            pl.BlockSpec((1, H), lambda j, i: (0, 0)),
            pl.BlockSpec((1, H), lambda j, i: (0, 0)),
            pl.BlockSpec((1, H), lambda j, i: (0, 0)),
            pl.BlockSpec((RB, H), lambda j, i: (i, 0)),
        ],
        out_specs=pl.BlockSpec((RB, H), lambda j, i: (i, 0)),
        out_shape=jax.ShapeDtypeStruct((N, H), jnp.float32),
        scratch_shapes=[
            pltpu.VMEM((1, H), jnp.float32),
            pltpu.VMEM((1, H), jnp.float32),
            pltpu.VMEM((1, H), jnp.float32),
            pltpu.VMEM((1, H), jnp.float32),
        ],
        compiler_params=pltpu.CompilerParams(
            dimension_semantics=("arbitrary", "arbitrary")),
    )(agg2, dinv, b2.reshape(1, H), gn2_w.reshape(1, H),
      gn2_b.reshape(1, H), gn2_a.reshape(1, H), x)


def kernel(x, edge_index, W1, b1, W2, b2, gn1_w, gn1_b, gn1_a, gn2_w, gn2_b, gn2_a):
    pad = jnp.full((2, EP - E), N, dtype=edge_index.dtype)
    ei = jnp.concatenate([edge_index, pad], axis=1)
    src = ei[0].reshape(NS, CH, CW)
    dst = ei[1].reshape(NS, CH, CW)
    zeros128 = jnp.zeros((NP, 128), jnp.float32)
    ones128 = jnp.ones((CW, 128), jnp.float32)

    degp = _deg_kernel(dst, zeros128, ones128)
    hs1, dinv = _tc_pre(x, W1, degp)
    agg1 = _conv_kernel(hs1, src, dst)
    hs2 = _tc_mid(agg1, dinv, b1, gn1_w, gn1_b, gn1_a, W2)
    agg2 = _conv_kernel(hs2, src, dst)
    return _tc_fin(agg2, dinv, b2, gn2_w, gn2_b, gn2_a, x)
